# Initial kernel scaffold; baseline (speedup 1.0000x reference)
import jax
import jax.numpy as jnp
from jax.experimental import pallas as pl


def kernel(x, node_type, edge_index, edge_type, landmark_mask, nt_table, in_proj_W, in_proj_b, gate_W1, gate_b1, gate_W2, gate_b2, c1_edge_emb, c1_msg_W1, c1_msg_b1, c1_msg_W2, c1_msg_b2, c1_att_W, c1_att_b, c2_edge_emb, c2_msg_W1, c2_msg_b1, c2_msg_W2, c2_msg_b2, c2_att_W, c2_att_b, fc_W, fc_b):
    raise NotImplementedError("write your pallas kernel here")


# speedup vs baseline: 7.8538x; 7.8538x over previous
import functools

import jax
import jax.numpy as jnp
from jax import lax
from jax.experimental import pallas as pl
from jax.experimental.pallas import tpu as pltpu
from jax.experimental.pallas import tpu_sc as plsc

N = 10000
E = 320000
HID = 128
NTYPE = 32
ETYPE = 16

NC = 2
NS = 16
NW = NC * NS
EPW = E // NW
BSZ = 128
NBAT = E // BSZ
BPW = -(-NBAT // NW)
NPAD = 10112
NEG_BIG = -3.0e38

@functools.lru_cache(maxsize=None)
def _mesh():
    return plsc.VectorSubcoreMesh(core_axis_name="c", subcore_axis_name="s",
                                  num_cores=NC, num_subcores=NS)


def _wid():
    return lax.axis_index("s") * NC + lax.axis_index("c")


def _prelude_body(x_ref, nt3_ref, wx_ref, tn_ref, b_ref, wa_ref, p_ref):
    nt = nt3_ref[0, 0, :]
    oh = (nt[:, None] == lax.broadcasted_iota(jnp.int32, (nt.shape[0], NTYPE), 1)
          ).astype(jnp.float32)
    h = jnp.dot(x_ref[...], wx_ref[...], preferred_element_type=jnp.float32)
    h = h + jnp.dot(oh, tn_ref[...], preferred_element_type=jnp.float32)
    h = jax.nn.relu(h + b_ref[...])
    p_ref[...] = jnp.dot(h, wa_ref[...], preferred_element_type=jnp.float32)


def _gate_body(nts_ref, ntd_ref, lms_ref, lmd_ref, ts_ref, td_ref, us_ref,
               ud_ref, b1_ref, w2t_ref, b2_ref, g_ref):
    nts = nts_ref[0, 0, :]
    ntd = ntd_ref[0, 0, :]
    be = nts.shape[0]
    iota = lax.broadcasted_iota(jnp.int32, (be, NTYPE), 1)
    ohs = (nts[:, None] == iota).astype(jnp.float32)
    ohd = (ntd[:, None] == iota).astype(jnp.float32)
    hid = jnp.dot(ohs, ts_ref[...], preferred_element_type=jnp.float32)
    hid = hid + jnp.dot(ohd, td_ref[...], preferred_element_type=jnp.float32)
    hid = hid + lms_ref[0, 0, :][:, None] * us_ref[...]
    hid = hid + lmd_ref[0, 0, :][:, None] * ud_ref[...]
    hid = jax.nn.relu(hid + b1_ref[...])
    grow = lax.dot_general(w2t_ref[...], hid, (((1,), (1,)), ((), ())),
                           preferred_element_type=jnp.float32)
    g_ref[...] = jax.nn.sigmoid(grow + b2_ref[...])[:, None, :]


def _edge_mlp_body(mpre_ref, et3_ref, qb_ref, w2_ref, b2_ref, attwt_ref,
                   attb_ref, m_ref, a_ref, gmax_ref, mx_sc):
    i = pl.program_id(0)
    ni = pl.num_programs(0)
    et = et3_ref[0, 0, :]
    oh = (et[:, None] == lax.broadcasted_iota(jnp.int32, (et.shape[0], ETYPE), 1)
          ).astype(jnp.float32)
    m1 = mpre_ref[...] + jnp.dot(oh, qb_ref[...], preferred_element_type=jnp.float32)
    m = jnp.dot(jax.nn.relu(m1), w2_ref[...], preferred_element_type=jnp.float32)
    m = m + b2_ref[...]
    m_ref[...] = m
    arow = lax.dot_general(attwt_ref[...], m, (((1,), (1,)), ((), ())),
                           preferred_element_type=jnp.float32)
    arow = arow + attb_ref[...]
    a_ref[...] = arow[:, None, :]
    bmax = jnp.max(arow)
    prev = jnp.where(i == 0, NEG_BIG, mx_sc[0, 0])
    mx_sc[0, 0] = jnp.maximum(prev, bmax)

    @pl.when(i == ni - 1)
    def _():
        gmax_ref[...] = jnp.full((8, 128), mx_sc[0, 0], dtype=jnp.float32)


def _combine_body(parts_ref, wa_ref, p_ref):
    h = jax.nn.relu(parts_ref[0] + parts_ref[1])
    p_ref[...] = jnp.dot(h, wa_ref[...], preferred_element_type=jnp.float32)


def _final_body(parts_ref, fcw_ref, fcb_ref, out_ref, acc):
    i = pl.program_id(0)
    ni = pl.num_programs(0)
    h = jax.nn.relu(parts_ref[0] + parts_ref[1])
    s = jnp.sum(h, axis=0, keepdims=True)

    @pl.when(i == 0)
    def _():
        acc[...] = s

    @pl.when(i != 0)
    def _():
        acc[...] = acc[...] + s

    @pl.when(i == ni - 1)
    def _():
        out_ref[...] = (jnp.dot(acc[...] * (1.0 / N), fcw_ref[...],
                                preferred_element_type=jnp.float32)
                        + fcb_ref[...])


def _sc_gate_gather_body(nt_hbm, lm_hbm, src_hbm, dst_hbm,
                         nts_out, ntd_out, lms_out, lmd_out,
                         ntb, lmb, srcb, dstb, ntsb, ntdb, lmsb, lmdb):
    wid = _wid()
    base = wid * EPW
    pltpu.sync_copy(nt_hbm, ntb)
    pltpu.sync_copy(lm_hbm, lmb)
    pltpu.sync_copy(src_hbm.at[pl.ds(base, EPW)], srcb)
    pltpu.sync_copy(dst_hbm.at[pl.ds(base, EPW)], dstb)

    def body(i, carry):
        sl = pl.ds(i * 16, 16)
        si = srcb[sl]
        di = dstb[sl]
        ntsb[sl] = plsc.load_gather(ntb, [si])
        lmsb[sl] = plsc.load_gather(lmb, [si])
        ntdb[sl] = plsc.load_gather(ntb, [di])
        lmdb[sl] = plsc.load_gather(lmb, [di])
        return carry

    lax.fori_loop(0, EPW // 16, body, 0)
    pltpu.sync_copy(ntsb, nts_out.at[pl.ds(base, EPW)])
    pltpu.sync_copy(ntdb, ntd_out.at[pl.ds(base, EPW)])
    pltpu.sync_copy(lmsb, lms_out.at[pl.ds(base, EPW)])
    pltpu.sync_copy(lmdb, lmd_out.at[pl.ds(base, EPW)])


def _sc_row_gather_body(p_hbm, src2_hbm, out_hbm, srcb, rows, sem):
    wid = _wid()

    def body(t, carry):
        b = t * NW + wid

        @pl.when(b < NBAT)
        def _():
            pltpu.sync_copy(src2_hbm.at[b], srcb)
            pltpu.async_copy(p_hbm.at[srcb], rows, sem).wait()
            pltpu.sync_copy(rows, out_hbm.at[pl.ds(b * BSZ, BSZ)])
        return carry

    lax.fori_loop(0, BPW, body, 0)


def _sc_denom_body(do_cnt, a_hbm, g_hbm, gmax_hbm, dst2_hbm,
                   eg_out, dpart_out, cpart_out,
                   ab, gb, egb, eb, dstb, onesb, gmaxb, zb, dacc, cacc):
    cid = lax.axis_index("c")
    sid = lax.axis_index("s")
    wid = sid * NC + cid

    def zbody(i, carry):
        zb[pl.ds(i * 16, 16)] = jnp.zeros((16,), jnp.float32)
        return carry
    lax.fori_loop(0, 40, zbody, 0)

    @pl.when(sid < 15)
    def _():
        pltpu.sync_copy(zb, dacc.at[pl.ds(sid * 640, 640)])
        if do_cnt:
            pltpu.sync_copy(zb, cacc.at[pl.ds(sid * 640, 640)])

    @pl.when(sid == 15)
    def _():
        pltpu.sync_copy(zb.at[pl.ds(0, 512)], dacc.at[pl.ds(9600, 512)])
        if do_cnt:
            pltpu.sync_copy(zb.at[pl.ds(0, 512)], cacc.at[pl.ds(9600, 512)])

    pltpu.sync_copy(gmax_hbm, gmaxb)
    if do_cnt:
        def obody(i, carry):
            onesb[pl.ds(i * 16, 16)] = jnp.ones((16,), jnp.float32)
            return carry
        lax.fori_loop(0, BSZ // 16, obody, 0)

    gmax = gmaxb[...]
    plsc.subcore_barrier()

    def sbody(t, carry):
        b = t * NW + wid

        @pl.when(b < NBAT)
        def _():
            o = b * BSZ
            pltpu.sync_copy(a_hbm.at[pl.ds(o, BSZ)], ab)
            pltpu.sync_copy(g_hbm.at[pl.ds(o, BSZ)], gb)
            pltpu.sync_copy(dst2_hbm.at[b], dstb)

            def ebody(i, carry2):
                sl = pl.ds(i * 16, 16)
                ev = jnp.exp(ab[sl] - gmax)
                eb[sl] = ev
                egb[sl] = ev * gb[sl]
                return carry2
            lax.fori_loop(0, BSZ // 16, ebody, 0)
            pltpu.sync_copy(egb, eg_out.at[pl.ds(o, BSZ)])
            pltpu.sync_copy(eb, dacc.at[dstb], add=True)
            if do_cnt:
                pltpu.sync_copy(onesb, cacc.at[dstb], add=True)
        return carry
    lax.fori_loop(0, BPW, sbody, 0)

    plsc.subcore_barrier()

    @pl.when(sid < 15)
    def _():
        pltpu.sync_copy(dacc.at[pl.ds(sid * 640, 640)],
                        dpart_out.at[cid, pl.ds(sid * 640, 640)])
        if do_cnt:
            pltpu.sync_copy(cacc.at[pl.ds(sid * 640, 640)],
                            cpart_out.at[cid, pl.ds(sid * 640, 640)])

    @pl.when(sid == 15)
    def _():
        pltpu.sync_copy(dacc.at[pl.ds(9600, 512)],
                        dpart_out.at[cid, pl.ds(9600, 512)])
        if do_cnt:
            pltpu.sync_copy(cacc.at[pl.ds(9600, 512)],
                            cpart_out.at[cid, pl.ds(9600, 512)])


def _sc_coeff_body(eg_hbm, dst_hbm, dpart_hbm, cpart_hbm, cf_out,
                   denb, cntb, tmp, egb, cfb, dstf):
    wid = _wid()
    base = wid * EPW

    pltpu.sync_copy(dpart_hbm.at[0], denb)
    pltpu.sync_copy(cpart_hbm.at[0], cntb)
    for part, accb in ((dpart_hbm, denb), (cpart_hbm, cntb)):
        for off, ln in ((0, 5120), (5120, 4992)):
            pltpu.sync_copy(part.at[1, pl.ds(off, ln)], tmp.at[pl.ds(0, ln)])

            def abody(i, carry2, off=off, accb=accb):
                s2 = pl.ds(off + i * 16, 16)
                accb[s2] = accb[s2] + tmp[pl.ds(i * 16, 16)]
                return carry2
            lax.fori_loop(0, ln // 16, abody, 0)

    pltpu.sync_copy(eg_hbm.at[pl.ds(base, EPW)], egb)
    pltpu.sync_copy(dst_hbm.at[pl.ds(base, EPW)], dstf)

    def cfbody(i, carry):
        sl = pl.ds(i * 16, 16)
        idx = dstf[sl]
        d16 = plsc.load_gather(denb, [idx])
        c16 = plsc.load_gather(cntb, [idx])
        cfb[sl] = egb[sl] / (d16 + 1e-16) / jnp.maximum(c16, 1.0)
        return carry
    lax.fori_loop(0, EPW // 16, cfbody, 0)
    pltpu.sync_copy(cfb, cf_out.at[pl.ds(base, EPW)])


def _sc_aggregate_body(m_hbm, cf_hbm, dst2_hbm, agg_out,
                       cfc, dstb, rows, aacc):
    cid = lax.axis_index("c")
    sid = lax.axis_index("s")
    wid = sid * NC + cid

    def zrows(i, carry):
        r = i // 8
        k = i % 8
        rows[r, pl.ds(k * 16, 16)] = jnp.zeros((16,), jnp.float32)
        return carry
    lax.fori_loop(0, BSZ * 8, zrows, 0)

    @pl.when(sid < 15)
    def _():
        for j in range(4):
            pltpu.sync_copy(rows, aacc.at[pl.ds(sid * 624 + j * 128, 128)])
        pltpu.sync_copy(rows.at[pl.ds(0, 112)],
                        aacc.at[pl.ds(sid * 624 + 512, 112)])

    @pl.when(sid == 15)
    def _():
        for j in range(5):
            pltpu.sync_copy(rows, aacc.at[pl.ds(9360 + j * 128, 128)])

    plsc.subcore_barrier()

    def mbody(t, carry):
        b = t * NW + wid

        @pl.when(b < NBAT)
        def _():
            o = b * BSZ
            pltpu.sync_copy(m_hbm.at[pl.ds(o, BSZ)], rows)
            pltpu.sync_copy(cf_hbm.at[pl.ds(o, BSZ)], cfc)
            pltpu.sync_copy(dst2_hbm.at[b], dstb)

            def scale_grp(jg, carry2):
                cf16 = cfc[pl.ds(jg * 16, 16)]
                for j in range(16):
                    sv = cf16[j]
                    r = jg * 16 + j
                    for kk in range(8):
                        sl = pl.ds(kk * 16, 16)
                        rows[r, sl] = rows[r, sl] * sv
                return carry2
            lax.fori_loop(0, BSZ // 16, scale_grp, 0)
            pltpu.sync_copy(rows, aacc.at[dstb], add=True)
        return carry
    lax.fori_loop(0, BPW, mbody, 0)

    plsc.subcore_barrier()

    @pl.when(sid < 15)
    def _():
        for j in range(4):
            pltpu.sync_copy(aacc.at[pl.ds(sid * 624 + j * 128, 128)],
                            agg_out.at[cid, pl.ds(sid * 624 + j * 128, 128)])
        pltpu.sync_copy(aacc.at[pl.ds(sid * 624 + 512, 112)],
                        agg_out.at[cid, pl.ds(sid * 624 + 512, 112)])

    @pl.when(sid == 15)
    def _():
        for j in range(5):
            pltpu.sync_copy(aacc.at[pl.ds(9360 + j * 128, 128)],
                            agg_out.at[cid, pl.ds(9360 + j * 128, 128)])


@functools.lru_cache(maxsize=None)
def _sc_kernels():
    mesh = _mesh()
    cp = pltpu.CompilerParams(needs_layout_passes=False)
    gate_gather = pl.kernel(
        _sc_gate_gather_body,
        out_type=(jax.ShapeDtypeStruct((E,), jnp.int32),
                  jax.ShapeDtypeStruct((E,), jnp.int32),
                  jax.ShapeDtypeStruct((E,), jnp.float32),
                  jax.ShapeDtypeStruct((E,), jnp.float32)),
        mesh=mesh,
        compiler_params=cp,
        scratch_types=[
            pltpu.VMEM((N,), jnp.int32),
            pltpu.VMEM((N,), jnp.float32),
            pltpu.VMEM((EPW,), jnp.int32),
            pltpu.VMEM((EPW,), jnp.int32),
            pltpu.VMEM((EPW,), jnp.int32),
            pltpu.VMEM((EPW,), jnp.int32),
            pltpu.VMEM((EPW,), jnp.float32),
            pltpu.VMEM((EPW,), jnp.float32),
        ],
    )
    row_gather = pl.kernel(
        _sc_row_gather_body,
        out_type=jax.ShapeDtypeStruct((E, HID), jnp.float32),
        mesh=mesh,
        compiler_params=cp,
        scratch_types=[
            pltpu.VMEM((BSZ,), jnp.int32),
            pltpu.VMEM((BSZ, HID), jnp.float32),
            pltpu.SemaphoreType.DMA,
        ],
    )

    def make_denom(do_cnt):
        return pl.kernel(
            functools.partial(_sc_denom_body, do_cnt),
            out_type=(jax.ShapeDtypeStruct((E,), jnp.float32),
                      jax.ShapeDtypeStruct((NC, NPAD), jnp.float32),
                      jax.ShapeDtypeStruct((NC, NPAD), jnp.float32)),
            mesh=mesh,
            compiler_params=cp,
            scratch_types=[
                pltpu.VMEM((BSZ,), jnp.float32),
                pltpu.VMEM((BSZ,), jnp.float32),
                pltpu.VMEM((BSZ,), jnp.float32),
                pltpu.VMEM((BSZ,), jnp.float32),
                pltpu.VMEM((BSZ,), jnp.int32),
                pltpu.VMEM((BSZ,), jnp.float32),
                pltpu.VMEM((16,), jnp.float32),
                pltpu.VMEM((640,), jnp.float32),
                pltpu.VMEM_SHARED((NPAD,), jnp.float32),
                pltpu.VMEM_SHARED((NPAD,), jnp.float32),
            ],
        )

    coeff = pl.kernel(
        _sc_coeff_body,
        out_type=jax.ShapeDtypeStruct((E,), jnp.float32),
        mesh=mesh,
        compiler_params=cp,
        scratch_types=[
            pltpu.VMEM((NPAD,), jnp.float32),
            pltpu.VMEM((NPAD,), jnp.float32),
            pltpu.VMEM((5120,), jnp.float32),
            pltpu.VMEM((EPW,), jnp.float32),
            pltpu.VMEM((EPW,), jnp.float32),
            pltpu.VMEM((EPW,), jnp.int32),
        ],
    )
    aggregate = pl.kernel(
        _sc_aggregate_body,
        out_type=jax.ShapeDtypeStruct((NC, N, HID), jnp.float32),
        mesh=mesh,
        compiler_params=cp,
        scratch_types=[
            pltpu.VMEM((BSZ,), jnp.float32),
            pltpu.VMEM((BSZ,), jnp.int32),
            pltpu.VMEM((BSZ, HID), jnp.float32),
            pltpu.VMEM_SHARED((N, HID), jnp.float32),
        ],
    )
    return gate_gather, row_gather, make_denom(True), make_denom(False), coeff, aggregate


_NBA = 1000
_BE = 2000


def _prelude(x, nt3, wx, tn, b, wa):
    return pl.pallas_call(
        _prelude_body,
        grid=(N // _NBA,),
        in_specs=[
            pl.BlockSpec((_NBA, HID), lambda i: (i, 0)),
            pl.BlockSpec((1, 1, _NBA), lambda i: (i, 0, 0)),
            pl.BlockSpec((HID, HID), lambda i: (0, 0)),
            pl.BlockSpec((NTYPE, HID), lambda i: (0, 0)),
            pl.BlockSpec((1, HID), lambda i: (0, 0)),
            pl.BlockSpec((HID, HID), lambda i: (0, 0)),
        ],
        out_specs=pl.BlockSpec((_NBA, HID), lambda i: (i, 0)),
        out_shape=jax.ShapeDtypeStruct((N, HID), jnp.float32),
    )(x, nt3, wx, tn, b, wa)


def _gate(nts3, ntd3, lms3, lmd3, ts, td, us, ud, b1, w2t, b2):
    gbl = E // _BE
    return pl.pallas_call(
        _gate_body,
        grid=(gbl,),
        in_specs=[
            pl.BlockSpec((1, 1, _BE), lambda i: (i, 0, 0)),
            pl.BlockSpec((1, 1, _BE), lambda i: (i, 0, 0)),
            pl.BlockSpec((1, 1, _BE), lambda i: (i, 0, 0)),
            pl.BlockSpec((1, 1, _BE), lambda i: (i, 0, 0)),
            pl.BlockSpec((NTYPE, NTYPE), lambda i: (0, 0)),
            pl.BlockSpec((NTYPE, NTYPE), lambda i: (0, 0)),
            pl.BlockSpec((1, NTYPE), lambda i: (0, 0)),
            pl.BlockSpec((1, NTYPE), lambda i: (0, 0)),
            pl.BlockSpec((1, NTYPE), lambda i: (0, 0)),
            pl.BlockSpec((1, NTYPE), lambda i: (0, 0)),
            pl.BlockSpec((1, 1), lambda i: (0, 0)),
        ],
        out_specs=pl.BlockSpec((1, 1, _BE), lambda i: (i, 0, 0)),
        out_shape=jax.ShapeDtypeStruct((gbl, 1, _BE), jnp.float32),
    )(nts3, ntd3, lms3, lmd3, ts, td, us, ud, b1, w2t, b2)


def _edge_mlp(mpre, et3, qb, w2, b2, attwt, attb):
    gbl = E // _BE
    return pl.pallas_call(
        _edge_mlp_body,
        grid=(gbl,),
        in_specs=[
            pl.BlockSpec((_BE, HID), lambda i: (i, 0)),
            pl.BlockSpec((1, 1, _BE), lambda i: (i, 0, 0)),
            pl.BlockSpec((ETYPE, HID), lambda i: (0, 0)),
            pl.BlockSpec((HID, HID), lambda i: (0, 0)),
            pl.BlockSpec((1, HID), lambda i: (0, 0)),
            pl.BlockSpec((1, HID), lambda i: (0, 0)),
            pl.BlockSpec((1, 1), lambda i: (0, 0)),
        ],
        out_specs=[
            pl.BlockSpec((_BE, HID), lambda i: (i, 0)),
            pl.BlockSpec((1, 1, _BE), lambda i: (i, 0, 0)),
            pl.BlockSpec((8, 128), lambda i: (0, 0)),
        ],
        out_shape=[
            jax.ShapeDtypeStruct((E, HID), jnp.float32),
            jax.ShapeDtypeStruct((gbl, 1, _BE), jnp.float32),
            jax.ShapeDtypeStruct((8, 128), jnp.float32),
        ],
        scratch_shapes=[pltpu.SMEM((1, 1), jnp.float32)],
    )(mpre, et3, qb, w2, b2, attwt, attb)


def _combine(parts, wa):
    return pl.pallas_call(
        _combine_body,
        grid=(N // _NBA,),
        in_specs=[
            pl.BlockSpec((NC, _NBA, HID), lambda i: (0, i, 0)),
            pl.BlockSpec((HID, HID), lambda i: (0, 0)),
        ],
        out_specs=pl.BlockSpec((_NBA, HID), lambda i: (i, 0)),
        out_shape=jax.ShapeDtypeStruct((N, HID), jnp.float32),
    )(parts, wa)


def _final(parts, fcw, fcb):
    return pl.pallas_call(
        _final_body,
        grid=(N // _NBA,),
        in_specs=[
            pl.BlockSpec((NC, _NBA, HID), lambda i: (0, i, 0)),
            pl.BlockSpec((HID, 16), lambda i: (0, 0)),
            pl.BlockSpec((1, 16), lambda i: (0, 0)),
        ],
        out_specs=pl.BlockSpec((1, 16), lambda i: (0, 0)),
        out_shape=jax.ShapeDtypeStruct((1, 16), jnp.float32),
        scratch_shapes=[pltpu.VMEM((1, HID), jnp.float32)],
    )(parts, fcw, fcb)


def kernel(x, node_type, edge_index, edge_type, landmark_mask, nt_table,
           in_proj_W, in_proj_b, gate_W1, gate_b1, gate_W2, gate_b2,
           c1_edge_emb, c1_msg_W1, c1_msg_b1, c1_msg_W2, c1_msg_b2,
           c1_att_W, c1_att_b, c2_edge_emb, c2_msg_W1, c2_msg_b1,
           c2_msg_W2, c2_msg_b2, c2_att_W, c2_att_b, fc_W, fc_b):
    f32 = jnp.float32
    node_type = node_type.astype(jnp.int32)
    edge_type = edge_type.astype(jnp.int32)
    src = edge_index[0].astype(jnp.int32)
    dst = edge_index[1].astype(jnp.int32)
    src2 = src.reshape(NBAT, BSZ)
    dst2 = dst.reshape(NBAT, BSZ)

    wx = in_proj_W[:HID]
    tn = jnp.dot(nt_table, in_proj_W[HID:], preferred_element_type=f32)
    b_in = in_proj_b[None, :]
    ts = jnp.dot(nt_table, gate_W1[0:8], preferred_element_type=f32)
    td = jnp.dot(nt_table, gate_W1[8:16], preferred_element_type=f32)
    us = gate_W1[16][None, :]
    ud = gate_W1[17][None, :]
    gb1 = gate_b1[None, :]
    w2t = gate_W2.T
    gb2 = gate_b2.reshape(1, 1)

    nt3 = node_type.reshape(N // _NBA, 1, _NBA)
    et3 = edge_type.reshape(E // _BE, 1, _BE)

    (_sc_gate_gather, _sc_row_gather, _sc_denom_cnt, _sc_denom_nocnt,
     _sc_coeff, _sc_aggregate) = _sc_kernels()

    nts, ntd, lms, lmd = _sc_gate_gather(node_type, landmark_mask, src, dst)
    gbl = E // _BE
    g = _gate(nts.reshape(gbl, 1, _BE), ntd.reshape(gbl, 1, _BE),
              lms.reshape(gbl, 1, _BE), lmd.reshape(gbl, 1, _BE),
              ts, td, us, ud, gb1, w2t, gb2).reshape(E)

    wa1 = c1_msg_W1[:HID]
    p1 = _prelude(x, nt3, wx, tn, b_in, wa1)

    def conv(p, edge_emb, msg_W1, msg_b1, msg_W2, msg_b2, att_W, att_b,
             do_cnt, cparts_prev):
        qb = jnp.dot(edge_emb, msg_W1[HID:], preferred_element_type=f32) \
            + msg_b1[None, :]
        mpre = _sc_row_gather(p, src2)
        m, a2, gmax8 = _edge_mlp(mpre, et3, qb, msg_W2, msg_b2[None, :],
                                 att_W.T, att_b.reshape(1, 1))
        gmax16 = gmax8.reshape(-1)[:16]
        if do_cnt:
            eg, dparts, cparts = _sc_denom_cnt(a2.reshape(E), g, gmax16, dst2)
        else:
            eg, dparts, _ = _sc_denom_nocnt(a2.reshape(E), g, gmax16, dst2)
            cparts = cparts_prev
        cf = _sc_coeff(eg, dst, dparts, cparts)
        agg = _sc_aggregate(m, cf, dst2)
        return agg, cparts

    agg1, cparts = conv(p1, c1_edge_emb, c1_msg_W1, c1_msg_b1, c1_msg_W2,
                        c1_msg_b2, c1_att_W, c1_att_b, True, None)
    p2 = _combine(agg1, c2_msg_W1[:HID])
    agg2, _ = conv(p2, c2_edge_emb, c2_msg_W1, c2_msg_b1, c2_msg_W2,
                   c2_msg_b2, c2_att_W, c2_att_b, False, cparts)
    return _final(agg2, fc_W, fc_b.reshape(1, 16))


# speedup vs baseline: 11.1290x; 1.4170x over previous
import functools

import jax
import jax.numpy as jnp
from jax import lax
from jax.experimental import pallas as pl
from jax.experimental.pallas import tpu as pltpu
from jax.experimental.pallas import tpu_sc as plsc

N = 10000
E = 320000
HID = 128
NTYPE = 32
ETYPE = 16

NC = 2
NS = 16
NW = NC * NS
EPW = E // NW
BSZ = 128
NBAT = E // BSZ
BPW = -(-NBAT // NW)
NPAD = 10112
NEG_BIG = -3.0e38

@functools.lru_cache(maxsize=None)
def _mesh():
    return plsc.VectorSubcoreMesh(core_axis_name="c", subcore_axis_name="s",
                                  num_cores=NC, num_subcores=NS)


def _wid():
    return lax.axis_index("s") * NC + lax.axis_index("c")


def _prelude_body(x_ref, nt3_ref, wx_ref, tn_ref, b_ref, wa_ref, p_ref):
    nt = nt3_ref[0, 0, :]
    oh = (nt[:, None] == lax.broadcasted_iota(jnp.int32, (nt.shape[0], NTYPE), 1)
          ).astype(jnp.float32)
    h = jnp.dot(x_ref[...], wx_ref[...], preferred_element_type=jnp.float32)
    h = h + jnp.dot(oh, tn_ref[...], preferred_element_type=jnp.float32)
    h = jax.nn.relu(h + b_ref[...])
    p_ref[...] = jnp.dot(h, wa_ref[...], preferred_element_type=jnp.float32)


def _gate_body(nts_ref, ntd_ref, lms_ref, lmd_ref, ts_ref, td_ref, us_ref,
               ud_ref, b1_ref, w2t_ref, b2_ref, g_ref):
    nts = nts_ref[0, 0, :]
    ntd = ntd_ref[0, 0, :]
    be = nts.shape[0]
    iota = lax.broadcasted_iota(jnp.int32, (be, NTYPE), 1)
    ohs = (nts[:, None] == iota).astype(jnp.float32)
    ohd = (ntd[:, None] == iota).astype(jnp.float32)
    hid = jnp.dot(ohs, ts_ref[...], preferred_element_type=jnp.float32)
    hid = hid + jnp.dot(ohd, td_ref[...], preferred_element_type=jnp.float32)
    hid = hid + lms_ref[0, 0, :][:, None] * us_ref[...]
    hid = hid + lmd_ref[0, 0, :][:, None] * ud_ref[...]
    hid = jax.nn.relu(hid + b1_ref[...])
    grow = lax.dot_general(w2t_ref[...], hid, (((1,), (1,)), ((), ())),
                           preferred_element_type=jnp.float32)
    g_ref[...] = jax.nn.sigmoid(grow + b2_ref[...])[:, None, :]


def _edge_mlp_body(mpre_ref, et3_ref, qb_ref, w2_ref, b2_ref, attwt_ref,
                   attb_ref, m_ref, a_ref, gmax_ref, mx_sc):
    i = pl.program_id(0)
    ni = pl.num_programs(0)
    et = et3_ref[0, 0, :]
    oh = (et[:, None] == lax.broadcasted_iota(jnp.int32, (et.shape[0], ETYPE), 1)
          ).astype(jnp.float32)
    m1 = mpre_ref[...] + jnp.dot(oh, qb_ref[...], preferred_element_type=jnp.float32)
    m = jnp.dot(jax.nn.relu(m1), w2_ref[...], preferred_element_type=jnp.float32)
    m = m + b2_ref[...]
    m_ref[...] = m
    arow = lax.dot_general(attwt_ref[...], m, (((1,), (1,)), ((), ())),
                           preferred_element_type=jnp.float32)
    arow = arow + attb_ref[...]
    a_ref[...] = arow[:, None, :]
    bmax = jnp.max(arow)
    prev = jnp.where(i == 0, NEG_BIG, mx_sc[0, 0])
    mx_sc[0, 0] = jnp.maximum(prev, bmax)

    @pl.when(i == ni - 1)
    def _():
        gmax_ref[...] = jnp.full((8, 128), mx_sc[0, 0], dtype=jnp.float32)


def _combine_body(parts_ref, wa_ref, p_ref):
    h = jax.nn.relu(parts_ref[0] + parts_ref[1])
    p_ref[...] = jnp.dot(h, wa_ref[...], preferred_element_type=jnp.float32)


def _final_body(parts_ref, fcw_ref, fcb_ref, out_ref, acc):
    i = pl.program_id(0)
    ni = pl.num_programs(0)
    h = jax.nn.relu(parts_ref[0] + parts_ref[1])
    s = jnp.sum(h, axis=0, keepdims=True)

    @pl.when(i == 0)
    def _():
        acc[...] = s

    @pl.when(i != 0)
    def _():
        acc[...] = acc[...] + s

    @pl.when(i == ni - 1)
    def _():
        out_ref[...] = (jnp.dot(acc[...] * (1.0 / N), fcw_ref[...],
                                preferred_element_type=jnp.float32)
                        + fcb_ref[...])


def _sc_gate_gather_body(nt_hbm, lm_hbm, src_hbm, dst_hbm,
                         nts_out, ntd_out, lms_out, lmd_out,
                         ntb, lmb, srcb, dstb, ntsb, ntdb, lmsb, lmdb):
    wid = _wid()
    base = wid * EPW
    pltpu.sync_copy(nt_hbm, ntb)
    pltpu.sync_copy(lm_hbm, lmb)
    pltpu.sync_copy(src_hbm.at[pl.ds(base, EPW)], srcb)
    pltpu.sync_copy(dst_hbm.at[pl.ds(base, EPW)], dstb)

    def body(i, carry):
        sl = pl.ds(i * 16, 16)
        si = srcb[sl]
        di = dstb[sl]
        ntsb[sl] = plsc.load_gather(ntb, [si])
        lmsb[sl] = plsc.load_gather(lmb, [si])
        ntdb[sl] = plsc.load_gather(ntb, [di])
        lmdb[sl] = plsc.load_gather(lmb, [di])
        return carry

    lax.fori_loop(0, EPW // 16, body, 0)
    pltpu.sync_copy(ntsb, nts_out.at[pl.ds(base, EPW)])
    pltpu.sync_copy(ntdb, ntd_out.at[pl.ds(base, EPW)])
    pltpu.sync_copy(lmsb, lms_out.at[pl.ds(base, EPW)])
    pltpu.sync_copy(lmdb, lmd_out.at[pl.ds(base, EPW)])


NCH = -(-EPW // BSZ)
TAIL = EPW - (NCH - 1) * BSZ


def _chunk(i):
    return i * BSZ, (BSZ if i < NCH - 1 else TAIL)


def _sc_row_gather_body(p_hbm, src_hbm, out_hbm, srcb,
                        rA, rB, gA, gB, oA, oB):
    wid = _wid()
    base = wid * EPW
    pltpu.sync_copy(src_hbm.at[pl.ds(base, EPW)], srcb)

    def body(t, carry):
        o0 = t * 2 * BSZ
        o1 = o0 + BSZ
        h0 = pltpu.async_copy(p_hbm.at[srcb.at[pl.ds(o0, BSZ)]], rA, gA)
        h1 = pltpu.async_copy(p_hbm.at[srcb.at[pl.ds(o1, BSZ)]], rB, gB)
        h0.wait()
        w0 = pltpu.async_copy(rA, out_hbm.at[pl.ds(base + o0, BSZ)], oA)
        h1.wait()
        w1 = pltpu.async_copy(rB, out_hbm.at[pl.ds(base + o1, BSZ)], oB)
        w0.wait()
        w1.wait()
        return carry

    lax.fori_loop(0, (NCH - 1) // 2, body, 0)
    off = (NCH - 1) * BSZ
    pltpu.async_copy(p_hbm.at[srcb.at[pl.ds(off, TAIL)]],
                     rA.at[pl.ds(0, TAIL)], gA).wait()
    pltpu.sync_copy(rA.at[pl.ds(0, TAIL)],
                    out_hbm.at[pl.ds(base + off, TAIL)])


def _sc_denom_body(do_cnt, a_hbm, g_hbm, gmax_hbm, dst_hbm,
                   eg_out, dpart_out, cpart_out,
                   ab, gb, egb, eb, dstb, onesb, gmaxb, zb, dacc, cacc):
    cid = lax.axis_index("c")
    sid = lax.axis_index("s")
    wid = sid * NC + cid
    base = wid * EPW

    def zbody(i, carry):
        zb[pl.ds(i * 16, 16)] = jnp.zeros((16,), jnp.float32)
        return carry
    lax.fori_loop(0, 40, zbody, 0)

    @pl.when(sid < 15)
    def _():
        pltpu.sync_copy(zb, dacc.at[pl.ds(sid * 640, 640)])
        if do_cnt:
            pltpu.sync_copy(zb, cacc.at[pl.ds(sid * 640, 640)])

    @pl.when(sid == 15)
    def _():
        pltpu.sync_copy(zb.at[pl.ds(0, 512)], dacc.at[pl.ds(9600, 512)])
        if do_cnt:
            pltpu.sync_copy(zb.at[pl.ds(0, 512)], cacc.at[pl.ds(9600, 512)])

    pltpu.sync_copy(gmax_hbm, gmaxb)
    pltpu.sync_copy(a_hbm.at[pl.ds(base, EPW)], ab)
    pltpu.sync_copy(g_hbm.at[pl.ds(base, EPW)], gb)
    pltpu.sync_copy(dst_hbm.at[pl.ds(base, EPW)], dstb)
    if do_cnt:
        def obody(i, carry):
            onesb[pl.ds(i * 16, 16)] = jnp.ones((16,), jnp.float32)
            return carry
        lax.fori_loop(0, BSZ // 16, obody, 0)

    gmax = gmaxb[...]

    def ebody(i, carry2):
        sl = pl.ds(i * 16, 16)
        ev = jnp.exp(ab[sl] - gmax)
        eb[sl] = ev
        egb[sl] = ev * gb[sl]
        return carry2
    lax.fori_loop(0, EPW // 16, ebody, 0)

    plsc.subcore_barrier()

    pltpu.sync_copy(egb, eg_out.at[pl.ds(base, EPW)])

    def sbody(i, carry):
        sl = pl.ds(i * BSZ, BSZ)
        idx = dstb.at[sl]
        pltpu.sync_copy(eb.at[sl], dacc.at[idx], add=True)
        if do_cnt:
            pltpu.sync_copy(onesb, cacc.at[idx], add=True)
        return carry
    lax.fori_loop(0, NCH - 1, sbody, 0)
    tsl = pl.ds((NCH - 1) * BSZ, TAIL)
    pltpu.sync_copy(eb.at[tsl], dacc.at[dstb.at[tsl]], add=True)
    if do_cnt:
        pltpu.sync_copy(onesb.at[pl.ds(0, TAIL)], cacc.at[dstb.at[tsl]],
                        add=True)

    plsc.subcore_barrier()

    @pl.when(sid < 15)
    def _():
        pltpu.sync_copy(dacc.at[pl.ds(sid * 640, 640)],
                        dpart_out.at[cid, pl.ds(sid * 640, 640)])
        if do_cnt:
            pltpu.sync_copy(cacc.at[pl.ds(sid * 640, 640)],
                            cpart_out.at[cid, pl.ds(sid * 640, 640)])

    @pl.when(sid == 15)
    def _():
        pltpu.sync_copy(dacc.at[pl.ds(9600, 512)],
                        dpart_out.at[cid, pl.ds(9600, 512)])
        if do_cnt:
            pltpu.sync_copy(cacc.at[pl.ds(9600, 512)],
                            cpart_out.at[cid, pl.ds(9600, 512)])


def _sc_coeff_body(eg_hbm, dst_hbm, dpart_hbm, cpart_hbm, cf_out,
                   denb, cntb, tmp, egb, cfb, dstf):
    wid = _wid()
    base = wid * EPW

    pltpu.sync_copy(dpart_hbm.at[0], denb)
    pltpu.sync_copy(cpart_hbm.at[0], cntb)
    for part, accb in ((dpart_hbm, denb), (cpart_hbm, cntb)):
        for off, ln in ((0, 5120), (5120, 4992)):
            pltpu.sync_copy(part.at[1, pl.ds(off, ln)], tmp.at[pl.ds(0, ln)])

            def abody(i, carry2, off=off, accb=accb):
                s2 = pl.ds(off + i * 16, 16)
                accb[s2] = accb[s2] + tmp[pl.ds(i * 16, 16)]
                return carry2
            lax.fori_loop(0, ln // 16, abody, 0)

    pltpu.sync_copy(eg_hbm.at[pl.ds(base, EPW)], egb)
    pltpu.sync_copy(dst_hbm.at[pl.ds(base, EPW)], dstf)

    def cfbody(i, carry):
        sl = pl.ds(i * 16, 16)
        idx = dstf[sl]
        d16 = plsc.load_gather(denb, [idx])
        c16 = plsc.load_gather(cntb, [idx])
        cfb[sl] = egb[sl] / (d16 + 1e-16) / jnp.maximum(c16, 1.0)
        return carry
    lax.fori_loop(0, EPW // 16, cfbody, 0)
    pltpu.sync_copy(cfb, cf_out.at[pl.ds(base, EPW)])


def _sc_aggregate_body(m_hbm, cf_hbm, dst_hbm, agg_out,
                       dstb, cf0, cf1, rows0, rows1, aacc,
                       m0, m1, c0, c1, s0, s1):
    cid = lax.axis_index("c")
    sid = lax.axis_index("s")
    wid = sid * NC + cid
    base = wid * EPW

    pltpu.sync_copy(dst_hbm.at[pl.ds(base, EPW)], dstb)

    def zrows(i, carry):
        r = i // 8
        k = i % 8
        rows0[r, pl.ds(k * 16, 16)] = jnp.zeros((16,), jnp.float32)
        return carry
    lax.fori_loop(0, BSZ * 8, zrows, 0)

    @pl.when(sid < 15)
    def _():
        for j in range(4):
            pltpu.sync_copy(rows0, aacc.at[pl.ds(sid * 624 + j * 128, 128)])
        pltpu.sync_copy(rows0.at[pl.ds(0, 112)],
                        aacc.at[pl.ds(sid * 624 + 512, 112)])

    @pl.when(sid == 15)
    def _():
        for j in range(5):
            pltpu.sync_copy(rows0, aacc.at[pl.ds(9360 + j * 128, 128)])

    plsc.subcore_barrier()

    def scale(rows, cfc, ngrp):
        def scale_grp(jg, carry2):
            cf16 = cfc[pl.ds(jg * 16, 16)]
            for j in range(16):
                sv = cf16[j]
                r = jg * 16 + j
                for kk in range(8):
                    sl = pl.ds(kk * 16, 16)
                    rows[r, sl] = rows[r, sl] * sv
            return carry2
        lax.fori_loop(0, ngrp, scale_grp, 0)

    def body(t, carry):
        o0 = t * 2 * BSZ
        o1 = o0 + BSZ
        mh0 = pltpu.async_copy(m_hbm.at[pl.ds(base + o0, BSZ)], rows0, m0)
        ch0 = pltpu.async_copy(cf_hbm.at[pl.ds(base + o0, BSZ)], cf0, c0)
        mh1 = pltpu.async_copy(m_hbm.at[pl.ds(base + o1, BSZ)], rows1, m1)
        ch1 = pltpu.async_copy(cf_hbm.at[pl.ds(base + o1, BSZ)], cf1, c1)
        mh0.wait()
        ch0.wait()
        scale(rows0, cf0, BSZ // 16)
        sh0 = pltpu.async_copy(rows0, aacc.at[dstb.at[pl.ds(o0, BSZ)]], s0,
                               add=True)
        mh1.wait()
        ch1.wait()
        scale(rows1, cf1, BSZ // 16)
        sh1 = pltpu.async_copy(rows1, aacc.at[dstb.at[pl.ds(o1, BSZ)]], s1,
                               add=True)
        sh0.wait()
        sh1.wait()
        return carry

    lax.fori_loop(0, (NCH - 1) // 2, body, 0)
    off = (NCH - 1) * BSZ
    pltpu.sync_copy(m_hbm.at[pl.ds(base + off, TAIL)],
                    rows0.at[pl.ds(0, TAIL)])
    pltpu.sync_copy(cf_hbm.at[pl.ds(base + off, TAIL)],
                    cf0.at[pl.ds(0, TAIL)])
    scale(rows0, cf0, TAIL // 16)
    pltpu.sync_copy(rows0.at[pl.ds(0, TAIL)],
                    aacc.at[dstb.at[pl.ds(off, TAIL)]], add=True)

    plsc.subcore_barrier()

    @pl.when(sid < 15)
    def _():
        for j in range(4):
            pltpu.sync_copy(aacc.at[pl.ds(sid * 624 + j * 128, 128)],
                            agg_out.at[cid, pl.ds(sid * 624 + j * 128, 128)])
        pltpu.sync_copy(aacc.at[pl.ds(sid * 624 + 512, 112)],
                        agg_out.at[cid, pl.ds(sid * 624 + 512, 112)])

    @pl.when(sid == 15)
    def _():
        for j in range(5):
            pltpu.sync_copy(aacc.at[pl.ds(9360 + j * 128, 128)],
                            agg_out.at[cid, pl.ds(9360 + j * 128, 128)])


@functools.lru_cache(maxsize=None)
def _sc_kernels():
    mesh = _mesh()
    cp = pltpu.CompilerParams(needs_layout_passes=False)
    gate_gather = pl.kernel(
        _sc_gate_gather_body,
        out_type=(jax.ShapeDtypeStruct((E,), jnp.int32),
                  jax.ShapeDtypeStruct((E,), jnp.int32),
                  jax.ShapeDtypeStruct((E,), jnp.float32),
                  jax.ShapeDtypeStruct((E,), jnp.float32)),
        mesh=mesh,
        compiler_params=cp,
        scratch_types=[
            pltpu.VMEM((N,), jnp.int32),
            pltpu.VMEM((N,), jnp.float32),
            pltpu.VMEM((EPW,), jnp.int32),
            pltpu.VMEM((EPW,), jnp.int32),
            pltpu.VMEM((EPW,), jnp.int32),
            pltpu.VMEM((EPW,), jnp.int32),
            pltpu.VMEM((EPW,), jnp.float32),
            pltpu.VMEM((EPW,), jnp.float32),
        ],
    )
    row_gather = pl.kernel(
        _sc_row_gather_body,
        out_type=jax.ShapeDtypeStruct((E, HID), jnp.float32),
        mesh=mesh,
        compiler_params=cp,
        scratch_types=(
            [pltpu.VMEM((EPW,), jnp.int32)]
            + [pltpu.VMEM((BSZ, HID), jnp.float32)] * 2
            + [pltpu.SemaphoreType.DMA] * 4
        ),
    )

    def make_denom(do_cnt):
        return pl.kernel(
            functools.partial(_sc_denom_body, do_cnt),
            out_type=(jax.ShapeDtypeStruct((E,), jnp.float32),
                      jax.ShapeDtypeStruct((NC, NPAD), jnp.float32),
                      jax.ShapeDtypeStruct((NC, NPAD), jnp.float32)),
            mesh=mesh,
            compiler_params=cp,
            scratch_types=[
                pltpu.VMEM((EPW,), jnp.float32),
                pltpu.VMEM((EPW,), jnp.float32),
                pltpu.VMEM((EPW,), jnp.float32),
                pltpu.VMEM((EPW,), jnp.float32),
                pltpu.VMEM((EPW,), jnp.int32),
                pltpu.VMEM((BSZ,), jnp.float32),
                pltpu.VMEM((16,), jnp.float32),
                pltpu.VMEM((640,), jnp.float32),
                pltpu.VMEM_SHARED((NPAD,), jnp.float32),
                pltpu.VMEM_SHARED((NPAD,), jnp.float32),
            ],
        )

    coeff = pl.kernel(
        _sc_coeff_body,
        out_type=jax.ShapeDtypeStruct((E,), jnp.float32),
        mesh=mesh,
        compiler_params=cp,
        scratch_types=[
            pltpu.VMEM((NPAD,), jnp.float32),
            pltpu.VMEM((NPAD,), jnp.float32),
            pltpu.VMEM((5120,), jnp.float32),
            pltpu.VMEM((EPW,), jnp.float32),
            pltpu.VMEM((EPW,), jnp.float32),
            pltpu.VMEM((EPW,), jnp.int32),
        ],
    )
    aggregate = pl.kernel(
        _sc_aggregate_body,
        out_type=jax.ShapeDtypeStruct((NC, N, HID), jnp.float32),
        mesh=mesh,
        compiler_params=cp,
        scratch_types=[
            pltpu.VMEM((EPW,), jnp.int32),
            pltpu.VMEM((BSZ,), jnp.float32),
            pltpu.VMEM((BSZ,), jnp.float32),
            pltpu.VMEM((BSZ, HID), jnp.float32),
            pltpu.VMEM((BSZ, HID), jnp.float32),
            pltpu.VMEM_SHARED((N, HID), jnp.float32),
        ] + [pltpu.SemaphoreType.DMA] * 6,
    )
    return gate_gather, row_gather, make_denom(True), make_denom(False), coeff, aggregate


_NBA = 1000
_BE = 2000


def _prelude(x, nt3, wx, tn, b, wa):
    return pl.pallas_call(
        _prelude_body,
        grid=(N // _NBA,),
        in_specs=[
            pl.BlockSpec((_NBA, HID), lambda i: (i, 0)),
            pl.BlockSpec((1, 1, _NBA), lambda i: (i, 0, 0)),
            pl.BlockSpec((HID, HID), lambda i: (0, 0)),
            pl.BlockSpec((NTYPE, HID), lambda i: (0, 0)),
            pl.BlockSpec((1, HID), lambda i: (0, 0)),
            pl.BlockSpec((HID, HID), lambda i: (0, 0)),
        ],
        out_specs=pl.BlockSpec((_NBA, HID), lambda i: (i, 0)),
        out_shape=jax.ShapeDtypeStruct((N, HID), jnp.float32),
    )(x, nt3, wx, tn, b, wa)


def _gate(nts3, ntd3, lms3, lmd3, ts, td, us, ud, b1, w2t, b2):
    gbl = E // _BE
    return pl.pallas_call(
        _gate_body,
        grid=(gbl,),
        in_specs=[
            pl.BlockSpec((1, 1, _BE), lambda i: (i, 0, 0)),
            pl.BlockSpec((1, 1, _BE), lambda i: (i, 0, 0)),
            pl.BlockSpec((1, 1, _BE), lambda i: (i, 0, 0)),
            pl.BlockSpec((1, 1, _BE), lambda i: (i, 0, 0)),
            pl.BlockSpec((NTYPE, NTYPE), lambda i: (0, 0)),
            pl.BlockSpec((NTYPE, NTYPE), lambda i: (0, 0)),
            pl.BlockSpec((1, NTYPE), lambda i: (0, 0)),
            pl.BlockSpec((1, NTYPE), lambda i: (0, 0)),
            pl.BlockSpec((1, NTYPE), lambda i: (0, 0)),
            pl.BlockSpec((1, NTYPE), lambda i: (0, 0)),
            pl.BlockSpec((1, 1), lambda i: (0, 0)),
        ],
        out_specs=pl.BlockSpec((1, 1, _BE), lambda i: (i, 0, 0)),
        out_shape=jax.ShapeDtypeStruct((gbl, 1, _BE), jnp.float32),
    )(nts3, ntd3, lms3, lmd3, ts, td, us, ud, b1, w2t, b2)


def _edge_mlp(mpre, et3, qb, w2, b2, attwt, attb):
    gbl = E // _BE
    return pl.pallas_call(
        _edge_mlp_body,
        grid=(gbl,),
        in_specs=[
            pl.BlockSpec((_BE, HID), lambda i: (i, 0)),
            pl.BlockSpec((1, 1, _BE), lambda i: (i, 0, 0)),
            pl.BlockSpec((ETYPE, HID), lambda i: (0, 0)),
            pl.BlockSpec((HID, HID), lambda i: (0, 0)),
            pl.BlockSpec((1, HID), lambda i: (0, 0)),
            pl.BlockSpec((1, HID), lambda i: (0, 0)),
            pl.BlockSpec((1, 1), lambda i: (0, 0)),
        ],
        out_specs=[
            pl.BlockSpec((_BE, HID), lambda i: (i, 0)),
            pl.BlockSpec((1, 1, _BE), lambda i: (i, 0, 0)),
            pl.BlockSpec((8, 128), lambda i: (0, 0)),
        ],
        out_shape=[
            jax.ShapeDtypeStruct((E, HID), jnp.float32),
            jax.ShapeDtypeStruct((gbl, 1, _BE), jnp.float32),
            jax.ShapeDtypeStruct((8, 128), jnp.float32),
        ],
        scratch_shapes=[pltpu.SMEM((1, 1), jnp.float32)],
    )(mpre, et3, qb, w2, b2, attwt, attb)


def _combine(parts, wa):
    return pl.pallas_call(
        _combine_body,
        grid=(N // _NBA,),
        in_specs=[
            pl.BlockSpec((NC, _NBA, HID), lambda i: (0, i, 0)),
            pl.BlockSpec((HID, HID), lambda i: (0, 0)),
        ],
        out_specs=pl.BlockSpec((_NBA, HID), lambda i: (i, 0)),
        out_shape=jax.ShapeDtypeStruct((N, HID), jnp.float32),
    )(parts, wa)


def _final(parts, fcw, fcb):
    return pl.pallas_call(
        _final_body,
        grid=(N // _NBA,),
        in_specs=[
            pl.BlockSpec((NC, _NBA, HID), lambda i: (0, i, 0)),
            pl.BlockSpec((HID, 16), lambda i: (0, 0)),
            pl.BlockSpec((1, 16), lambda i: (0, 0)),
        ],
        out_specs=pl.BlockSpec((1, 16), lambda i: (0, 0)),
        out_shape=jax.ShapeDtypeStruct((1, 16), jnp.float32),
        scratch_shapes=[pltpu.VMEM((1, HID), jnp.float32)],
    )(parts, fcw, fcb)


def kernel(x, node_type, edge_index, edge_type, landmark_mask, nt_table,
           in_proj_W, in_proj_b, gate_W1, gate_b1, gate_W2, gate_b2,
           c1_edge_emb, c1_msg_W1, c1_msg_b1, c1_msg_W2, c1_msg_b2,
           c1_att_W, c1_att_b, c2_edge_emb, c2_msg_W1, c2_msg_b1,
           c2_msg_W2, c2_msg_b2, c2_att_W, c2_att_b, fc_W, fc_b):
    f32 = jnp.float32
    node_type = node_type.astype(jnp.int32)
    edge_type = edge_type.astype(jnp.int32)
    src = edge_index[0].astype(jnp.int32)
    dst = edge_index[1].astype(jnp.int32)

    wx = in_proj_W[:HID]
    tn = jnp.dot(nt_table, in_proj_W[HID:], preferred_element_type=f32)
    b_in = in_proj_b[None, :]
    ts = jnp.dot(nt_table, gate_W1[0:8], preferred_element_type=f32)
    td = jnp.dot(nt_table, gate_W1[8:16], preferred_element_type=f32)
    us = gate_W1[16][None, :]
    ud = gate_W1[17][None, :]
    gb1 = gate_b1[None, :]
    w2t = gate_W2.T
    gb2 = gate_b2.reshape(1, 1)

    nt3 = node_type.reshape(N // _NBA, 1, _NBA)
    et3 = edge_type.reshape(E // _BE, 1, _BE)

    (_sc_gate_gather, _sc_row_gather, _sc_denom_cnt, _sc_denom_nocnt,
     _sc_coeff, _sc_aggregate) = _sc_kernels()

    nts, ntd, lms, lmd = _sc_gate_gather(node_type, landmark_mask, src, dst)
    gbl = E // _BE
    g = _gate(nts.reshape(gbl, 1, _BE), ntd.reshape(gbl, 1, _BE),
              lms.reshape(gbl, 1, _BE), lmd.reshape(gbl, 1, _BE),
              ts, td, us, ud, gb1, w2t, gb2).reshape(E)

    wa1 = c1_msg_W1[:HID]
    p1 = _prelude(x, nt3, wx, tn, b_in, wa1)

    def conv(p, edge_emb, msg_W1, msg_b1, msg_W2, msg_b2, att_W, att_b,
             do_cnt, cparts_prev):
        qb = jnp.dot(edge_emb, msg_W1[HID:], preferred_element_type=f32) \
            + msg_b1[None, :]
        mpre = _sc_row_gather(p, src)
        m, a2, gmax8 = _edge_mlp(mpre, et3, qb, msg_W2, msg_b2[None, :],
                                 att_W.T, att_b.reshape(1, 1))
        gmax16 = gmax8.reshape(-1)[:16]
        if do_cnt:
            eg, dparts, cparts = _sc_denom_cnt(a2.reshape(E), g, gmax16, dst)
        else:
            eg, dparts, _ = _sc_denom_nocnt(a2.reshape(E), g, gmax16, dst)
            cparts = cparts_prev
        cf = _sc_coeff(eg, dst, dparts, cparts)
        agg = _sc_aggregate(m, cf, dst)
        return agg, cparts

    agg1, cparts = conv(p1, c1_edge_emb, c1_msg_W1, c1_msg_b1, c1_msg_W2,
                        c1_msg_b2, c1_att_W, c1_att_b, True, None)
    p2 = _combine(agg1, c2_msg_W1[:HID])
    agg2, _ = conv(p2, c2_edge_emb, c2_msg_W1, c2_msg_b1, c2_msg_W2,
                   c2_msg_b2, c2_att_W, c2_att_b, False, cparts)
    return _final(agg2, fc_W, fc_b.reshape(1, 16))


# speedup vs baseline: 11.2120x; 1.0075x over previous
import functools

import jax
import jax.numpy as jnp
from jax import lax
from jax.experimental import pallas as pl
from jax.experimental.pallas import tpu as pltpu
from jax.experimental.pallas import tpu_sc as plsc

N = 10000
E = 320000
HID = 128
NTYPE = 32
ETYPE = 16

NC = 2
NS = 16
NW = NC * NS
EPW = E // NW
BSZ = 128
NBAT = E // BSZ
BPW = -(-NBAT // NW)
NPAD = 10112
NEG_BIG = -3.0e38

@functools.lru_cache(maxsize=None)
def _mesh():
    return plsc.VectorSubcoreMesh(core_axis_name="c", subcore_axis_name="s",
                                  num_cores=NC, num_subcores=NS)


def _wid():
    return lax.axis_index("s") * NC + lax.axis_index("c")


def _prelude_body(x_ref, nt3_ref, wx_ref, tn_ref, b_ref, wa_ref, p_ref):
    nt = nt3_ref[0, 0, :]
    oh = (nt[:, None] == lax.broadcasted_iota(jnp.int32, (nt.shape[0], NTYPE), 1)
          ).astype(jnp.float32)
    h = jnp.dot(x_ref[...], wx_ref[...], preferred_element_type=jnp.float32)
    h = h + jnp.dot(oh, tn_ref[...], preferred_element_type=jnp.float32)
    h = jax.nn.relu(h + b_ref[...])
    p_ref[...] = jnp.dot(h, wa_ref[...], preferred_element_type=jnp.float32)


def _gate_body(nts_ref, ntd_ref, lms_ref, lmd_ref, ts_ref, td_ref, us_ref,
               ud_ref, b1_ref, w2t_ref, b2_ref, g_ref):
    nts = nts_ref[0, 0, :]
    ntd = ntd_ref[0, 0, :]
    be = nts.shape[0]
    iota = lax.broadcasted_iota(jnp.int32, (be, NTYPE), 1)
    ohs = (nts[:, None] == iota).astype(jnp.float32)
    ohd = (ntd[:, None] == iota).astype(jnp.float32)
    hid = jnp.dot(ohs, ts_ref[...], preferred_element_type=jnp.float32)
    hid = hid + jnp.dot(ohd, td_ref[...], preferred_element_type=jnp.float32)
    hid = hid + lms_ref[0, 0, :][:, None] * us_ref[...]
    hid = hid + lmd_ref[0, 0, :][:, None] * ud_ref[...]
    hid = jax.nn.relu(hid + b1_ref[...])
    grow = lax.dot_general(w2t_ref[...], hid, (((1,), (1,)), ((), ())),
                           preferred_element_type=jnp.float32)
    g_ref[...] = jax.nn.sigmoid(grow + b2_ref[...])[:, None, :]


def _edge_mlp_body(mpre_ref, et3_ref, qb_ref, wtil_ref, c_ref,
                   m_ref, a_ref, gmax_ref, mx_sc):
    i = pl.program_id(0)
    ni = pl.num_programs(0)
    et = et3_ref[0, 0, :]
    oh = (et[:, None] == lax.broadcasted_iota(jnp.int32, (et.shape[0], ETYPE), 1)
          ).astype(jnp.float32)
    m1 = mpre_ref[...] + jnp.dot(oh, qb_ref[...], preferred_element_type=jnp.float32)
    r = jax.nn.relu(m1)
    m_ref[...] = r
    arow = lax.dot_general(wtil_ref[...], r, (((1,), (1,)), ((), ())),
                           preferred_element_type=jnp.float32)
    arow = arow + c_ref[...]
    a_ref[...] = arow[:, None, :]
    bmax = jnp.max(arow)
    prev = jnp.where(i == 0, NEG_BIG, mx_sc[0, 0])
    mx_sc[0, 0] = jnp.maximum(prev, bmax)

    @pl.when(i == ni - 1)
    def _():
        gmax_ref[...] = jnp.full((8, 128), mx_sc[0, 0], dtype=jnp.float32)


def _node_h(parts_ref, sp3_ref, w2_ref, b2_ref):
    s = sp3_ref[0, 0, :] + sp3_ref[0, 1, :]
    h = jnp.dot(parts_ref[0] + parts_ref[1], w2_ref[...],
                preferred_element_type=jnp.float32)
    return jax.nn.relu(h + s[:, None] * b2_ref[...])


def _combine_body(parts_ref, sp3_ref, w2_ref, b2_ref, wa_ref, p_ref):
    h = _node_h(parts_ref, sp3_ref, w2_ref, b2_ref)
    p_ref[...] = jnp.dot(h, wa_ref[...], preferred_element_type=jnp.float32)


def _final_body(parts_ref, sp3_ref, w2_ref, b2_ref, fcw_ref, fcb_ref,
                out_ref, acc):
    i = pl.program_id(0)
    ni = pl.num_programs(0)
    h = _node_h(parts_ref, sp3_ref, w2_ref, b2_ref)
    s = jnp.sum(h, axis=0, keepdims=True)

    @pl.when(i == 0)
    def _():
        acc[...] = s

    @pl.when(i != 0)
    def _():
        acc[...] = acc[...] + s

    @pl.when(i == ni - 1)
    def _():
        out_ref[...] = (jnp.dot(acc[...] * (1.0 / N), fcw_ref[...],
                                preferred_element_type=jnp.float32)
                        + fcb_ref[...])


def _sc_gate_gather_body(nt_hbm, lm_hbm, src_hbm, dst_hbm,
                         nts_out, ntd_out, lms_out, lmd_out,
                         ntb, lmb, srcb, dstb, ntsb, ntdb, lmsb, lmdb):
    wid = _wid()
    base = wid * EPW
    pltpu.sync_copy(nt_hbm, ntb)
    pltpu.sync_copy(lm_hbm, lmb)
    pltpu.sync_copy(src_hbm.at[pl.ds(base, EPW)], srcb)
    pltpu.sync_copy(dst_hbm.at[pl.ds(base, EPW)], dstb)

    def body(i, carry):
        sl = pl.ds(i * 16, 16)
        si = srcb[sl]
        di = dstb[sl]
        ntsb[sl] = plsc.load_gather(ntb, [si])
        lmsb[sl] = plsc.load_gather(lmb, [si])
        ntdb[sl] = plsc.load_gather(ntb, [di])
        lmdb[sl] = plsc.load_gather(lmb, [di])
        return carry

    lax.fori_loop(0, EPW // 16, body, 0)
    pltpu.sync_copy(ntsb, nts_out.at[pl.ds(base, EPW)])
    pltpu.sync_copy(ntdb, ntd_out.at[pl.ds(base, EPW)])
    pltpu.sync_copy(lmsb, lms_out.at[pl.ds(base, EPW)])
    pltpu.sync_copy(lmdb, lmd_out.at[pl.ds(base, EPW)])


NCH = -(-EPW // BSZ)
TAIL = EPW - (NCH - 1) * BSZ


def _chunk(i):
    return i * BSZ, (BSZ if i < NCH - 1 else TAIL)


def _sc_row_gather_body(p_hbm, src_hbm, out_hbm, srcb,
                        rA, rB, gA, gB, oA, oB):
    wid = _wid()
    base = wid * EPW
    pltpu.sync_copy(src_hbm.at[pl.ds(base, EPW)], srcb)

    def body(t, carry):
        o0 = t * 2 * BSZ
        o1 = o0 + BSZ
        h0 = pltpu.async_copy(p_hbm.at[srcb.at[pl.ds(o0, BSZ)]], rA, gA)
        h1 = pltpu.async_copy(p_hbm.at[srcb.at[pl.ds(o1, BSZ)]], rB, gB)
        h0.wait()
        w0 = pltpu.async_copy(rA, out_hbm.at[pl.ds(base + o0, BSZ)], oA)
        h1.wait()
        w1 = pltpu.async_copy(rB, out_hbm.at[pl.ds(base + o1, BSZ)], oB)
        w0.wait()
        w1.wait()
        return carry

    lax.fori_loop(0, (NCH - 1) // 2, body, 0)
    off = (NCH - 1) * BSZ
    pltpu.async_copy(p_hbm.at[srcb.at[pl.ds(off, TAIL)]],
                     rA.at[pl.ds(0, TAIL)], gA).wait()
    pltpu.sync_copy(rA.at[pl.ds(0, TAIL)],
                    out_hbm.at[pl.ds(base + off, TAIL)])


def _sc_denom_body(do_cnt, a_hbm, g_hbm, gmax_hbm, dst_hbm,
                   eg_out, dpart_out, cpart_out,
                   ab, gb, egb, eb, dstb, onesb, gmaxb, zb, dacc, cacc):
    cid = lax.axis_index("c")
    sid = lax.axis_index("s")
    wid = sid * NC + cid
    base = wid * EPW

    def zbody(i, carry):
        zb[pl.ds(i * 16, 16)] = jnp.zeros((16,), jnp.float32)
        return carry
    lax.fori_loop(0, 40, zbody, 0)

    @pl.when(sid < 15)
    def _():
        pltpu.sync_copy(zb, dacc.at[pl.ds(sid * 640, 640)])
        if do_cnt:
            pltpu.sync_copy(zb, cacc.at[pl.ds(sid * 640, 640)])

    @pl.when(sid == 15)
    def _():
        pltpu.sync_copy(zb.at[pl.ds(0, 512)], dacc.at[pl.ds(9600, 512)])
        if do_cnt:
            pltpu.sync_copy(zb.at[pl.ds(0, 512)], cacc.at[pl.ds(9600, 512)])

    pltpu.sync_copy(gmax_hbm, gmaxb)
    pltpu.sync_copy(a_hbm.at[pl.ds(base, EPW)], ab)
    pltpu.sync_copy(g_hbm.at[pl.ds(base, EPW)], gb)
    pltpu.sync_copy(dst_hbm.at[pl.ds(base, EPW)], dstb)
    if do_cnt:
        def obody(i, carry):
            onesb[pl.ds(i * 16, 16)] = jnp.ones((16,), jnp.float32)
            return carry
        lax.fori_loop(0, BSZ // 16, obody, 0)

    gmax = gmaxb[...]

    def ebody(i, carry2):
        sl = pl.ds(i * 16, 16)
        ev = jnp.exp(ab[sl] - gmax)
        eb[sl] = ev
        egb[sl] = ev * gb[sl]
        return carry2
    lax.fori_loop(0, EPW // 16, ebody, 0)

    plsc.subcore_barrier()

    pltpu.sync_copy(egb, eg_out.at[pl.ds(base, EPW)])

    def sbody(i, carry):
        sl = pl.ds(i * BSZ, BSZ)
        idx = dstb.at[sl]
        pltpu.sync_copy(eb.at[sl], dacc.at[idx], add=True)
        if do_cnt:
            pltpu.sync_copy(onesb, cacc.at[idx], add=True)
        return carry
    lax.fori_loop(0, NCH - 1, sbody, 0)
    tsl = pl.ds((NCH - 1) * BSZ, TAIL)
    pltpu.sync_copy(eb.at[tsl], dacc.at[dstb.at[tsl]], add=True)
    if do_cnt:
        pltpu.sync_copy(onesb.at[pl.ds(0, TAIL)], cacc.at[dstb.at[tsl]],
                        add=True)

    plsc.subcore_barrier()

    @pl.when(sid < 15)
    def _():
        pltpu.sync_copy(dacc.at[pl.ds(sid * 640, 640)],
                        dpart_out.at[cid, pl.ds(sid * 640, 640)])
        if do_cnt:
            pltpu.sync_copy(cacc.at[pl.ds(sid * 640, 640)],
                            cpart_out.at[cid, pl.ds(sid * 640, 640)])

    @pl.when(sid == 15)
    def _():
        pltpu.sync_copy(dacc.at[pl.ds(9600, 512)],
                        dpart_out.at[cid, pl.ds(9600, 512)])
        if do_cnt:
            pltpu.sync_copy(cacc.at[pl.ds(9600, 512)],
                            cpart_out.at[cid, pl.ds(9600, 512)])


def _sc_coeff_body(eg_hbm, dst_hbm, dpart_hbm, cpart_hbm, cf_out,
                   denb, cntb, tmp, egb, cfb, dstf):
    wid = _wid()
    base = wid * EPW

    pltpu.sync_copy(dpart_hbm.at[0], denb)
    pltpu.sync_copy(cpart_hbm.at[0], cntb)
    for part, accb in ((dpart_hbm, denb), (cpart_hbm, cntb)):
        for off, ln in ((0, 5120), (5120, 4992)):
            pltpu.sync_copy(part.at[1, pl.ds(off, ln)], tmp.at[pl.ds(0, ln)])

            def abody(i, carry2, off=off, accb=accb):
                s2 = pl.ds(off + i * 16, 16)
                accb[s2] = accb[s2] + tmp[pl.ds(i * 16, 16)]
                return carry2
            lax.fori_loop(0, ln // 16, abody, 0)

    pltpu.sync_copy(eg_hbm.at[pl.ds(base, EPW)], egb)
    pltpu.sync_copy(dst_hbm.at[pl.ds(base, EPW)], dstf)

    def cfbody(i, carry):
        sl = pl.ds(i * 16, 16)
        idx = dstf[sl]
        d16 = plsc.load_gather(denb, [idx])
        c16 = plsc.load_gather(cntb, [idx])
        cfb[sl] = egb[sl] / (d16 + 1e-16) / jnp.maximum(c16, 1.0)
        return carry
    lax.fori_loop(0, EPW // 16, cfbody, 0)
    pltpu.sync_copy(cfb, cf_out.at[pl.ds(base, EPW)])


def _sc_aggregate_body(m_hbm, cf_hbm, dst_hbm, agg_out, sparts_out,
                       dstb, cf0, cf1, rows0, rows1, zb, aacc, sacc,
                       m0, m1, c0, c1, s0, s1, q0, q1):
    cid = lax.axis_index("c")
    sid = lax.axis_index("s")
    wid = sid * NC + cid
    base = wid * EPW

    pltpu.sync_copy(dst_hbm.at[pl.ds(base, EPW)], dstb)

    def zbody(i, carry):
        zb[pl.ds(i * 16, 16)] = jnp.zeros((16,), jnp.float32)
        return carry
    lax.fori_loop(0, 40, zbody, 0)

    @pl.when(sid < 15)
    def _():
        pltpu.sync_copy(zb, sacc.at[pl.ds(sid * 640, 640)])

    @pl.when(sid == 15)
    def _():
        pltpu.sync_copy(zb.at[pl.ds(0, 512)], sacc.at[pl.ds(9600, 512)])

    def zrows(i, carry):
        r = i // 8
        k = i % 8
        rows0[r, pl.ds(k * 16, 16)] = jnp.zeros((16,), jnp.float32)
        return carry
    lax.fori_loop(0, BSZ * 8, zrows, 0)

    @pl.when(sid < 15)
    def _():
        for j in range(4):
            pltpu.sync_copy(rows0, aacc.at[pl.ds(sid * 624 + j * 128, 128)])
        pltpu.sync_copy(rows0.at[pl.ds(0, 112)],
                        aacc.at[pl.ds(sid * 624 + 512, 112)])

    @pl.when(sid == 15)
    def _():
        for j in range(5):
            pltpu.sync_copy(rows0, aacc.at[pl.ds(9360 + j * 128, 128)])

    plsc.subcore_barrier()

    def scale(rows, cfc, ngrp):
        def scale_grp(jg, carry2):
            cf16 = cfc[pl.ds(jg * 16, 16)]
            for j in range(16):
                sv = cf16[j]
                r = jg * 16 + j
                for kk in range(8):
                    sl = pl.ds(kk * 16, 16)
                    rows[r, sl] = rows[r, sl] * sv
            return carry2
        lax.fori_loop(0, ngrp, scale_grp, 0)

    def body(t, carry):
        o0 = t * 2 * BSZ
        o1 = o0 + BSZ
        mh0 = pltpu.async_copy(m_hbm.at[pl.ds(base + o0, BSZ)], rows0, m0)
        ch0 = pltpu.async_copy(cf_hbm.at[pl.ds(base + o0, BSZ)], cf0, c0)
        mh1 = pltpu.async_copy(m_hbm.at[pl.ds(base + o1, BSZ)], rows1, m1)
        ch1 = pltpu.async_copy(cf_hbm.at[pl.ds(base + o1, BSZ)], cf1, c1)
        mh0.wait()
        ch0.wait()
        scale(rows0, cf0, BSZ // 16)
        sh0 = pltpu.async_copy(rows0, aacc.at[dstb.at[pl.ds(o0, BSZ)]], s0,
                               add=True)
        qh0 = pltpu.async_copy(cf0, sacc.at[dstb.at[pl.ds(o0, BSZ)]], q0,
                               add=True)
        mh1.wait()
        ch1.wait()
        scale(rows1, cf1, BSZ // 16)
        sh1 = pltpu.async_copy(rows1, aacc.at[dstb.at[pl.ds(o1, BSZ)]], s1,
                               add=True)
        qh1 = pltpu.async_copy(cf1, sacc.at[dstb.at[pl.ds(o1, BSZ)]], q1,
                               add=True)
        sh0.wait()
        qh0.wait()
        sh1.wait()
        qh1.wait()
        return carry

    lax.fori_loop(0, (NCH - 1) // 2, body, 0)
    off = (NCH - 1) * BSZ
    pltpu.sync_copy(m_hbm.at[pl.ds(base + off, TAIL)],
                    rows0.at[pl.ds(0, TAIL)])
    pltpu.sync_copy(cf_hbm.at[pl.ds(base + off, TAIL)],
                    cf0.at[pl.ds(0, TAIL)])
    scale(rows0, cf0, TAIL // 16)
    pltpu.sync_copy(rows0.at[pl.ds(0, TAIL)],
                    aacc.at[dstb.at[pl.ds(off, TAIL)]], add=True)
    pltpu.sync_copy(cf0.at[pl.ds(0, TAIL)],
                    sacc.at[dstb.at[pl.ds(off, TAIL)]], add=True)

    plsc.subcore_barrier()

    @pl.when(sid < 15)
    def _():
        for j in range(4):
            pltpu.sync_copy(aacc.at[pl.ds(sid * 624 + j * 128, 128)],
                            agg_out.at[cid, pl.ds(sid * 624 + j * 128, 128)])
        pltpu.sync_copy(aacc.at[pl.ds(sid * 624 + 512, 112)],
                        agg_out.at[cid, pl.ds(sid * 624 + 512, 112)])
        pltpu.sync_copy(sacc.at[pl.ds(sid * 640, 640)],
                        sparts_out.at[cid, pl.ds(sid * 640, 640)])

    @pl.when(sid == 15)
    def _():
        for j in range(5):
            pltpu.sync_copy(aacc.at[pl.ds(9360 + j * 128, 128)],
                            agg_out.at[cid, pl.ds(9360 + j * 128, 128)])
        pltpu.sync_copy(sacc.at[pl.ds(9600, 512)],
                        sparts_out.at[cid, pl.ds(9600, 512)])


@functools.lru_cache(maxsize=None)
def _sc_kernels():
    mesh = _mesh()
    cp = pltpu.CompilerParams(needs_layout_passes=False)
    gate_gather = pl.kernel(
        _sc_gate_gather_body,
        out_type=(jax.ShapeDtypeStruct((E,), jnp.int32),
                  jax.ShapeDtypeStruct((E,), jnp.int32),
                  jax.ShapeDtypeStruct((E,), jnp.float32),
                  jax.ShapeDtypeStruct((E,), jnp.float32)),
        mesh=mesh,
        compiler_params=cp,
        scratch_types=[
            pltpu.VMEM((N,), jnp.int32),
            pltpu.VMEM((N,), jnp.float32),
            pltpu.VMEM((EPW,), jnp.int32),
            pltpu.VMEM((EPW,), jnp.int32),
            pltpu.VMEM((EPW,), jnp.int32),
            pltpu.VMEM((EPW,), jnp.int32),
            pltpu.VMEM((EPW,), jnp.float32),
            pltpu.VMEM((EPW,), jnp.float32),
        ],
    )
    row_gather = pl.kernel(
        _sc_row_gather_body,
        out_type=jax.ShapeDtypeStruct((E, HID), jnp.float32),
        mesh=mesh,
        compiler_params=cp,
        scratch_types=(
            [pltpu.VMEM((EPW,), jnp.int32)]
            + [pltpu.VMEM((BSZ, HID), jnp.float32)] * 2
            + [pltpu.SemaphoreType.DMA] * 4
        ),
    )

    def make_denom(do_cnt):
        return pl.kernel(
            functools.partial(_sc_denom_body, do_cnt),
            out_type=(jax.ShapeDtypeStruct((E,), jnp.float32),
                      jax.ShapeDtypeStruct((NC, NPAD), jnp.float32),
                      jax.ShapeDtypeStruct((NC, NPAD), jnp.float32)),
            mesh=mesh,
            compiler_params=cp,
            scratch_types=[
                pltpu.VMEM((EPW,), jnp.float32),
                pltpu.VMEM((EPW,), jnp.float32),
                pltpu.VMEM((EPW,), jnp.float32),
                pltpu.VMEM((EPW,), jnp.float32),
                pltpu.VMEM((EPW,), jnp.int32),
                pltpu.VMEM((BSZ,), jnp.float32),
                pltpu.VMEM((16,), jnp.float32),
                pltpu.VMEM((640,), jnp.float32),
                pltpu.VMEM_SHARED((NPAD,), jnp.float32),
                pltpu.VMEM_SHARED((NPAD,), jnp.float32),
            ],
        )

    coeff = pl.kernel(
        _sc_coeff_body,
        out_type=jax.ShapeDtypeStruct((E,), jnp.float32),
        mesh=mesh,
        compiler_params=cp,
        scratch_types=[
            pltpu.VMEM((NPAD,), jnp.float32),
            pltpu.VMEM((NPAD,), jnp.float32),
            pltpu.VMEM((5120,), jnp.float32),
            pltpu.VMEM((EPW,), jnp.float32),
            pltpu.VMEM((EPW,), jnp.float32),
            pltpu.VMEM((EPW,), jnp.int32),
        ],
    )
    aggregate = pl.kernel(
        _sc_aggregate_body,
        out_type=(jax.ShapeDtypeStruct((NC, N, HID), jnp.float32),
                  jax.ShapeDtypeStruct((NC, NPAD), jnp.float32)),
        mesh=mesh,
        compiler_params=cp,
        scratch_types=[
            pltpu.VMEM((EPW,), jnp.int32),
            pltpu.VMEM((BSZ,), jnp.float32),
            pltpu.VMEM((BSZ,), jnp.float32),
            pltpu.VMEM((BSZ, HID), jnp.float32),
            pltpu.VMEM((BSZ, HID), jnp.float32),
            pltpu.VMEM((640,), jnp.float32),
            pltpu.VMEM_SHARED((N, HID), jnp.float32),
            pltpu.VMEM_SHARED((NPAD,), jnp.float32),
        ] + [pltpu.SemaphoreType.DMA] * 8,
    )
    return gate_gather, row_gather, make_denom(True), make_denom(False), coeff, aggregate


_NBA = 1000
_BE = 2000


def _prelude(x, nt3, wx, tn, b, wa):
    return pl.pallas_call(
        _prelude_body,
        grid=(N // _NBA,),
        in_specs=[
            pl.BlockSpec((_NBA, HID), lambda i: (i, 0)),
            pl.BlockSpec((1, 1, _NBA), lambda i: (i, 0, 0)),
            pl.BlockSpec((HID, HID), lambda i: (0, 0)),
            pl.BlockSpec((NTYPE, HID), lambda i: (0, 0)),
            pl.BlockSpec((1, HID), lambda i: (0, 0)),
            pl.BlockSpec((HID, HID), lambda i: (0, 0)),
        ],
        out_specs=pl.BlockSpec((_NBA, HID), lambda i: (i, 0)),
        out_shape=jax.ShapeDtypeStruct((N, HID), jnp.float32),
    )(x, nt3, wx, tn, b, wa)


def _gate(nts3, ntd3, lms3, lmd3, ts, td, us, ud, b1, w2t, b2):
    gbl = E // _BE
    return pl.pallas_call(
        _gate_body,
        grid=(gbl,),
        in_specs=[
            pl.BlockSpec((1, 1, _BE), lambda i: (i, 0, 0)),
            pl.BlockSpec((1, 1, _BE), lambda i: (i, 0, 0)),
            pl.BlockSpec((1, 1, _BE), lambda i: (i, 0, 0)),
            pl.BlockSpec((1, 1, _BE), lambda i: (i, 0, 0)),
            pl.BlockSpec((NTYPE, NTYPE), lambda i: (0, 0)),
            pl.BlockSpec((NTYPE, NTYPE), lambda i: (0, 0)),
            pl.BlockSpec((1, NTYPE), lambda i: (0, 0)),
            pl.BlockSpec((1, NTYPE), lambda i: (0, 0)),
            pl.BlockSpec((1, NTYPE), lambda i: (0, 0)),
            pl.BlockSpec((1, NTYPE), lambda i: (0, 0)),
            pl.BlockSpec((1, 1), lambda i: (0, 0)),
        ],
        out_specs=pl.BlockSpec((1, 1, _BE), lambda i: (i, 0, 0)),
        out_shape=jax.ShapeDtypeStruct((gbl, 1, _BE), jnp.float32),
    )(nts3, ntd3, lms3, lmd3, ts, td, us, ud, b1, w2t, b2)


def _edge_mlp(mpre, et3, qb, wtil, c):
    gbl = E // _BE
    return pl.pallas_call(
        _edge_mlp_body,
        grid=(gbl,),
        in_specs=[
            pl.BlockSpec((_BE, HID), lambda i: (i, 0)),
            pl.BlockSpec((1, 1, _BE), lambda i: (i, 0, 0)),
            pl.BlockSpec((ETYPE, HID), lambda i: (0, 0)),
            pl.BlockSpec((1, HID), lambda i: (0, 0)),
            pl.BlockSpec((1, 1), lambda i: (0, 0)),
        ],
        out_specs=[
            pl.BlockSpec((_BE, HID), lambda i: (i, 0)),
            pl.BlockSpec((1, 1, _BE), lambda i: (i, 0, 0)),
            pl.BlockSpec((8, 128), lambda i: (0, 0)),
        ],
        out_shape=[
            jax.ShapeDtypeStruct((E, HID), jnp.float32),
            jax.ShapeDtypeStruct((gbl, 1, _BE), jnp.float32),
            jax.ShapeDtypeStruct((8, 128), jnp.float32),
        ],
        scratch_shapes=[pltpu.SMEM((1, 1), jnp.float32)],
    )(mpre, et3, qb, wtil, c)


def _combine(parts, sp3, w2, b2, wa):
    return pl.pallas_call(
        _combine_body,
        grid=(N // _NBA,),
        in_specs=[
            pl.BlockSpec((NC, _NBA, HID), lambda i: (0, i, 0)),
            pl.BlockSpec((1, NC, _NBA), lambda i: (i, 0, 0)),
            pl.BlockSpec((HID, HID), lambda i: (0, 0)),
            pl.BlockSpec((1, HID), lambda i: (0, 0)),
            pl.BlockSpec((HID, HID), lambda i: (0, 0)),
        ],
        out_specs=pl.BlockSpec((_NBA, HID), lambda i: (i, 0)),
        out_shape=jax.ShapeDtypeStruct((N, HID), jnp.float32),
    )(parts, sp3, w2, b2, wa)


def _final(parts, sp3, w2, b2, fcw, fcb):
    return pl.pallas_call(
        _final_body,
        grid=(N // _NBA,),
        in_specs=[
            pl.BlockSpec((NC, _NBA, HID), lambda i: (0, i, 0)),
            pl.BlockSpec((1, NC, _NBA), lambda i: (i, 0, 0)),
            pl.BlockSpec((HID, HID), lambda i: (0, 0)),
            pl.BlockSpec((1, HID), lambda i: (0, 0)),
            pl.BlockSpec((HID, 16), lambda i: (0, 0)),
            pl.BlockSpec((1, 16), lambda i: (0, 0)),
        ],
        out_specs=pl.BlockSpec((1, 16), lambda i: (0, 0)),
        out_shape=jax.ShapeDtypeStruct((1, 16), jnp.float32),
        scratch_shapes=[pltpu.VMEM((1, HID), jnp.float32)],
    )(parts, sp3, w2, b2, fcw, fcb)


def kernel(x, node_type, edge_index, edge_type, landmark_mask, nt_table,
           in_proj_W, in_proj_b, gate_W1, gate_b1, gate_W2, gate_b2,
           c1_edge_emb, c1_msg_W1, c1_msg_b1, c1_msg_W2, c1_msg_b2,
           c1_att_W, c1_att_b, c2_edge_emb, c2_msg_W1, c2_msg_b1,
           c2_msg_W2, c2_msg_b2, c2_att_W, c2_att_b, fc_W, fc_b):
    f32 = jnp.float32
    node_type = node_type.astype(jnp.int32)
    edge_type = edge_type.astype(jnp.int32)
    src = edge_index[0].astype(jnp.int32)
    dst = edge_index[1].astype(jnp.int32)

    wx = in_proj_W[:HID]
    tn = jnp.dot(nt_table, in_proj_W[HID:], preferred_element_type=f32)
    b_in = in_proj_b[None, :]
    ts = jnp.dot(nt_table, gate_W1[0:8], preferred_element_type=f32)
    td = jnp.dot(nt_table, gate_W1[8:16], preferred_element_type=f32)
    us = gate_W1[16][None, :]
    ud = gate_W1[17][None, :]
    gb1 = gate_b1[None, :]
    w2t = gate_W2.T
    gb2 = gate_b2.reshape(1, 1)

    nt3 = node_type.reshape(N // _NBA, 1, _NBA)
    et3 = edge_type.reshape(E // _BE, 1, _BE)

    (_sc_gate_gather, _sc_row_gather, _sc_denom_cnt, _sc_denom_nocnt,
     _sc_coeff, _sc_aggregate) = _sc_kernels()

    nts, ntd, lms, lmd = _sc_gate_gather(node_type, landmark_mask, src, dst)
    gbl = E // _BE
    g = _gate(nts.reshape(gbl, 1, _BE), ntd.reshape(gbl, 1, _BE),
              lms.reshape(gbl, 1, _BE), lmd.reshape(gbl, 1, _BE),
              ts, td, us, ud, gb1, w2t, gb2).reshape(E)

    wa1 = c1_msg_W1[:HID]
    p1 = _prelude(x, nt3, wx, tn, b_in, wa1)

    def conv(p, edge_emb, msg_W1, msg_b1, msg_W2, msg_b2, att_W, att_b,
             do_cnt, cparts_prev):
        qb = jnp.dot(edge_emb, msg_W1[HID:], preferred_element_type=f32) \
            + msg_b1[None, :]
        wtil = jnp.dot(msg_W2, att_W, preferred_element_type=f32).T
        cconst = (jnp.dot(msg_b2[None, :], att_W, preferred_element_type=f32)
                  + att_b).reshape(1, 1)
        mpre = _sc_row_gather(p, src)
        r, a2, gmax8 = _edge_mlp(mpre, et3, qb, wtil, cconst)
        gmax16 = gmax8.reshape(-1)[:16]
        if do_cnt:
            eg, dparts, cparts = _sc_denom_cnt(a2.reshape(E), g, gmax16, dst)
        else:
            eg, dparts, _ = _sc_denom_nocnt(a2.reshape(E), g, gmax16, dst)
            cparts = cparts_prev
        cf = _sc_coeff(eg, dst, dparts, cparts)
        agg, sparts = _sc_aggregate(r, cf, dst)
        sp3 = sparts[:, :N].reshape(NC, N // _NBA, _NBA).transpose(1, 0, 2)
        return agg, sp3, cparts

    agg1, sp31, cparts = conv(p1, c1_edge_emb, c1_msg_W1, c1_msg_b1,
                              c1_msg_W2, c1_msg_b2, c1_att_W, c1_att_b,
                              True, None)
    p2 = _combine(agg1, sp31, c1_msg_W2, c1_msg_b2[None, :], c2_msg_W1[:HID])
    agg2, sp32, _ = conv(p2, c2_edge_emb, c2_msg_W1, c2_msg_b1, c2_msg_W2,
                         c2_msg_b2, c2_att_W, c2_att_b, False, cparts)
    return _final(agg2, sp32, c2_msg_W2, c2_msg_b2[None, :], fc_W,
                  fc_b.reshape(1, 16))


# speedup vs baseline: 11.5159x; 1.0271x over previous
import functools

import jax
import jax.numpy as jnp
from jax import lax
from jax.experimental import pallas as pl
from jax.experimental.pallas import tpu as pltpu
from jax.experimental.pallas import tpu_sc as plsc

N = 10000
E = 320000
HID = 128
NTYPE = 32
ETYPE = 16

NC = 2
NS = 16
NW = NC * NS
EPW = E // NW
BSZ = 128
NBAT = E // BSZ
BPW = -(-NBAT // NW)
NPAD = 10112
NEG_BIG = -3.0e38

@functools.lru_cache(maxsize=None)
def _mesh():
    return plsc.VectorSubcoreMesh(core_axis_name="c", subcore_axis_name="s",
                                  num_cores=NC, num_subcores=NS)


def _wid():
    return lax.axis_index("s") * NC + lax.axis_index("c")


def _prelude_body(x_ref, nt3_ref, wx_ref, tn_ref, b_ref, wa_ref, p_ref):
    nt = nt3_ref[0, 0, :]
    oh = (nt[:, None] == lax.broadcasted_iota(jnp.int32, (nt.shape[0], NTYPE), 1)
          ).astype(jnp.float32)
    h = jnp.dot(x_ref[...], wx_ref[...], preferred_element_type=jnp.float32)
    h = h + jnp.dot(oh, tn_ref[...], preferred_element_type=jnp.float32)
    h = jax.nn.relu(h + b_ref[...])
    p_ref[...] = jnp.dot(h, wa_ref[...], preferred_element_type=jnp.float32)


def _gate_body(nts_ref, ntd_ref, lms_ref, lmd_ref, ts_ref, td_ref, us_ref,
               ud_ref, b1_ref, w2t_ref, b2_ref, g_ref):
    nts = nts_ref[0, 0, :]
    ntd = ntd_ref[0, 0, :]
    be = nts.shape[0]
    iota = lax.broadcasted_iota(jnp.int32, (be, NTYPE), 1)
    ohs = (nts[:, None] == iota).astype(jnp.float32)
    ohd = (ntd[:, None] == iota).astype(jnp.float32)
    hid = jnp.dot(ohs, ts_ref[...], preferred_element_type=jnp.float32)
    hid = hid + jnp.dot(ohd, td_ref[...], preferred_element_type=jnp.float32)
    hid = hid + lms_ref[0, 0, :][:, None] * us_ref[...]
    hid = hid + lmd_ref[0, 0, :][:, None] * ud_ref[...]
    hid = jax.nn.relu(hid + b1_ref[...])
    grow = lax.dot_general(w2t_ref[...], hid, (((1,), (1,)), ((), ())),
                           preferred_element_type=jnp.float32)
    g_ref[...] = jax.nn.sigmoid(grow + b2_ref[...])[:, None, :]


def _edge_mlp_body(mpre_ref, et3_ref, qb_ref, wtil_ref, c_ref,
                   m_ref, a_ref, gmax_ref, mx_sc):
    i = pl.program_id(0)
    ni = pl.num_programs(0)
    et = et3_ref[0, 0, :]
    oh = (et[:, None] == lax.broadcasted_iota(jnp.int32, (et.shape[0], ETYPE), 1)
          ).astype(jnp.float32)
    m1 = mpre_ref[...] + jnp.dot(oh, qb_ref[...], preferred_element_type=jnp.float32)
    r = jax.nn.relu(m1)
    m_ref[...] = r
    arow = lax.dot_general(wtil_ref[...], r, (((1,), (1,)), ((), ())),
                           preferred_element_type=jnp.float32)
    arow = arow + c_ref[...]
    a_ref[...] = arow[:, None, :]
    bmax = jnp.max(arow)
    prev = jnp.where(i == 0, NEG_BIG, mx_sc[0, 0])
    mx_sc[0, 0] = jnp.maximum(prev, bmax)

    @pl.when(i == ni - 1)
    def _():
        gmax_ref[...] = jnp.full((8, 128), mx_sc[0, 0], dtype=jnp.float32)


def _node_h(parts_ref, sp3_ref, w2_ref, b2_ref):
    s = sp3_ref[0, 0, :] + sp3_ref[0, 1, :]
    h = jnp.dot(parts_ref[0] + parts_ref[1], w2_ref[...],
                preferred_element_type=jnp.float32)
    return jax.nn.relu(h + s[:, None] * b2_ref[...])


def _combine_body(parts_ref, sp3_ref, w2_ref, b2_ref, wa_ref, p_ref):
    h = _node_h(parts_ref, sp3_ref, w2_ref, b2_ref)
    p_ref[...] = jnp.dot(h, wa_ref[...], preferred_element_type=jnp.float32)


def _final_body(parts_ref, sp3_ref, w2_ref, b2_ref, fcw_ref, fcb_ref,
                out_ref, acc):
    i = pl.program_id(0)
    ni = pl.num_programs(0)
    h = _node_h(parts_ref, sp3_ref, w2_ref, b2_ref)
    s = jnp.sum(h, axis=0, keepdims=True)

    @pl.when(i == 0)
    def _():
        acc[...] = s

    @pl.when(i != 0)
    def _():
        acc[...] = acc[...] + s

    @pl.when(i == ni - 1)
    def _():
        out_ref[...] = (jnp.dot(acc[...] * (1.0 / N), fcw_ref[...],
                                preferred_element_type=jnp.float32)
                        + fcb_ref[...])


def _sc_gate_gather_body(nt_hbm, lm_hbm, src_hbm, dst_hbm,
                         nts_out, ntd_out, lms_out, lmd_out,
                         ntb, lmb, srcb, dstb, ntsb, ntdb, lmsb, lmdb):
    wid = _wid()
    base = wid * EPW
    pltpu.sync_copy(nt_hbm, ntb)
    pltpu.sync_copy(lm_hbm, lmb)
    pltpu.sync_copy(src_hbm.at[pl.ds(base, EPW)], srcb)
    pltpu.sync_copy(dst_hbm.at[pl.ds(base, EPW)], dstb)

    def body(i, carry):
        sl = pl.ds(i * 16, 16)
        si = srcb[sl]
        di = dstb[sl]
        ntsb[sl] = plsc.load_gather(ntb, [si])
        lmsb[sl] = plsc.load_gather(lmb, [si])
        ntdb[sl] = plsc.load_gather(ntb, [di])
        lmdb[sl] = plsc.load_gather(lmb, [di])
        return carry

    lax.fori_loop(0, EPW // 16, body, 0)
    pltpu.sync_copy(ntsb, nts_out.at[pl.ds(base, EPW)])
    pltpu.sync_copy(ntdb, ntd_out.at[pl.ds(base, EPW)])
    pltpu.sync_copy(lmsb, lms_out.at[pl.ds(base, EPW)])
    pltpu.sync_copy(lmdb, lmd_out.at[pl.ds(base, EPW)])


NCH = -(-EPW // BSZ)
TAIL = EPW - (NCH - 1) * BSZ


def _chunk(i):
    return i * BSZ, (BSZ if i < NCH - 1 else TAIL)


def _sc_row_gather_body(p_hbm, src_hbm, out_hbm, srcb,
                        rA, rB, rC, rD, gA, gB, gC, gD, oA, oB, oC, oD):
    wid = _wid()
    base = wid * EPW
    pltpu.sync_copy(src_hbm.at[pl.ds(base, EPW)], srcb)
    bufs = (rA, rB, rC, rD)
    gsem = (gA, gB, gC, gD)
    osem = (oA, oB, oC, oD)

    def body(t, carry):
        o0 = t * 4 * BSZ
        gh = [pltpu.async_copy(
            p_hbm.at[srcb.at[pl.ds(o0 + s * BSZ, BSZ)]], bufs[s], gsem[s])
            for s in range(4)]
        oh = []
        for s in range(4):
            gh[s].wait()
            oh.append(pltpu.async_copy(
                bufs[s], out_hbm.at[pl.ds(base + o0 + s * BSZ, BSZ)],
                osem[s]))
        for s in range(4):
            oh[s].wait()
        return carry

    nquad = (NCH - 1) // 4
    lax.fori_loop(0, nquad, body, 0)
    o0 = nquad * 4 * BSZ
    gh = [pltpu.async_copy(
        p_hbm.at[srcb.at[pl.ds(o0 + s * BSZ, BSZ)]], bufs[s], gsem[s])
        for s in range((NCH - 1) % 4)]
    oh = []
    for s in range((NCH - 1) % 4):
        gh[s].wait()
        oh.append(pltpu.async_copy(
            bufs[s], out_hbm.at[pl.ds(base + o0 + s * BSZ, BSZ)], osem[s]))
    for h in oh:
        h.wait()
    off = (NCH - 1) * BSZ
    pltpu.async_copy(p_hbm.at[srcb.at[pl.ds(off, TAIL)]],
                     rA.at[pl.ds(0, TAIL)], gA).wait()
    pltpu.sync_copy(rA.at[pl.ds(0, TAIL)],
                    out_hbm.at[pl.ds(base + off, TAIL)])


def _sc_denom_body(do_cnt, a_hbm, g_hbm, gmax_hbm, dst_hbm,
                   eg_out, dpart_out, cpart_out,
                   ab, gb, egb, eb, dstb, onesb, gmaxb, zb, dacc, cacc):
    cid = lax.axis_index("c")
    sid = lax.axis_index("s")
    wid = sid * NC + cid
    base = wid * EPW

    def zbody(i, carry):
        zb[pl.ds(i * 16, 16)] = jnp.zeros((16,), jnp.float32)
        return carry
    lax.fori_loop(0, 40, zbody, 0)

    @pl.when(sid < 15)
    def _():
        pltpu.sync_copy(zb, dacc.at[pl.ds(sid * 640, 640)])
        if do_cnt:
            pltpu.sync_copy(zb, cacc.at[pl.ds(sid * 640, 640)])

    @pl.when(sid == 15)
    def _():
        pltpu.sync_copy(zb.at[pl.ds(0, 512)], dacc.at[pl.ds(9600, 512)])
        if do_cnt:
            pltpu.sync_copy(zb.at[pl.ds(0, 512)], cacc.at[pl.ds(9600, 512)])

    pltpu.sync_copy(gmax_hbm, gmaxb)
    pltpu.sync_copy(a_hbm.at[pl.ds(base, EPW)], ab)
    pltpu.sync_copy(g_hbm.at[pl.ds(base, EPW)], gb)
    pltpu.sync_copy(dst_hbm.at[pl.ds(base, EPW)], dstb)
    if do_cnt:
        def obody(i, carry):
            onesb[pl.ds(i * 16, 16)] = jnp.ones((16,), jnp.float32)
            return carry
        lax.fori_loop(0, BSZ // 16, obody, 0)

    gmax = gmaxb[...]

    def ebody(i, carry2):
        sl = pl.ds(i * 16, 16)
        ev = jnp.exp(ab[sl] - gmax)
        eb[sl] = ev
        egb[sl] = ev * gb[sl]
        return carry2
    lax.fori_loop(0, EPW // 16, ebody, 0)

    plsc.subcore_barrier()

    pltpu.sync_copy(egb, eg_out.at[pl.ds(base, EPW)])

    def sbody(i, carry):
        sl = pl.ds(i * BSZ, BSZ)
        idx = dstb.at[sl]
        pltpu.sync_copy(eb.at[sl], dacc.at[idx], add=True)
        if do_cnt:
            pltpu.sync_copy(onesb, cacc.at[idx], add=True)
        return carry
    lax.fori_loop(0, NCH - 1, sbody, 0)
    tsl = pl.ds((NCH - 1) * BSZ, TAIL)
    pltpu.sync_copy(eb.at[tsl], dacc.at[dstb.at[tsl]], add=True)
    if do_cnt:
        pltpu.sync_copy(onesb.at[pl.ds(0, TAIL)], cacc.at[dstb.at[tsl]],
                        add=True)

    plsc.subcore_barrier()

    @pl.when(sid < 15)
    def _():
        pltpu.sync_copy(dacc.at[pl.ds(sid * 640, 640)],
                        dpart_out.at[cid, pl.ds(sid * 640, 640)])
        if do_cnt:
            pltpu.sync_copy(cacc.at[pl.ds(sid * 640, 640)],
                            cpart_out.at[cid, pl.ds(sid * 640, 640)])

    @pl.when(sid == 15)
    def _():
        pltpu.sync_copy(dacc.at[pl.ds(9600, 512)],
                        dpart_out.at[cid, pl.ds(9600, 512)])
        if do_cnt:
            pltpu.sync_copy(cacc.at[pl.ds(9600, 512)],
                            cpart_out.at[cid, pl.ds(9600, 512)])


def _sc_coeff_body(eg_hbm, dst_hbm, dpart_hbm, cpart_hbm, cf_out,
                   denb, cntb, tmp, egb, cfb, dstf):
    wid = _wid()
    base = wid * EPW

    pltpu.sync_copy(dpart_hbm.at[0], denb)
    pltpu.sync_copy(cpart_hbm.at[0], cntb)
    for part, accb in ((dpart_hbm, denb), (cpart_hbm, cntb)):
        for off, ln in ((0, 5120), (5120, 4992)):
            pltpu.sync_copy(part.at[1, pl.ds(off, ln)], tmp.at[pl.ds(0, ln)])

            def abody(i, carry2, off=off, accb=accb):
                s2 = pl.ds(off + i * 16, 16)
                accb[s2] = accb[s2] + tmp[pl.ds(i * 16, 16)]
                return carry2
            lax.fori_loop(0, ln // 16, abody, 0)

    pltpu.sync_copy(eg_hbm.at[pl.ds(base, EPW)], egb)
    pltpu.sync_copy(dst_hbm.at[pl.ds(base, EPW)], dstf)

    def cfbody(i, carry):
        sl = pl.ds(i * 16, 16)
        idx = dstf[sl]
        d16 = plsc.load_gather(denb, [idx])
        c16 = plsc.load_gather(cntb, [idx])
        cfb[sl] = egb[sl] / (d16 + 1e-16) / jnp.maximum(c16, 1.0)
        return carry
    lax.fori_loop(0, EPW // 16, cfbody, 0)
    pltpu.sync_copy(cfb, cf_out.at[pl.ds(base, EPW)])


def _sc_aggregate_body(m_hbm, cf_hbm, dst_hbm, agg_out, sparts_out,
                       cfA, cfB, cfC, dcA, dcB, dcC, rA, rB, rC, aacc, sacc,
                       mA, mB, mC, cA, cB, cC, dA, dB, dC,
                       sA, sB, sC, qA, qB, qC):
    cid = lax.axis_index("c")
    sid = lax.axis_index("s")
    wid = sid * NC + cid
    base = wid * EPW

    def zcf(i, carry):
        cfA[pl.ds(i * 16, 16)] = jnp.zeros((16,), jnp.float32)
        return carry
    lax.fori_loop(0, BSZ // 16, zcf, 0)

    @pl.when(sid < 15)
    def _():
        for j in range(5):
            pltpu.sync_copy(cfA, sacc.at[pl.ds(sid * 640 + j * 128, 128)])

    @pl.when(sid == 15)
    def _():
        for j in range(4):
            pltpu.sync_copy(cfA, sacc.at[pl.ds(9600 + j * 128, 128)])

    def zrows(i, carry):
        r = i // 8
        k = i % 8
        rA[r, pl.ds(k * 16, 16)] = jnp.zeros((16,), jnp.float32)
        return carry
    lax.fori_loop(0, BSZ * 8, zrows, 0)

    @pl.when(sid < 15)
    def _():
        for j in range(4):
            pltpu.sync_copy(rA, aacc.at[pl.ds(sid * 624 + j * 128, 128)])
        pltpu.sync_copy(rA.at[pl.ds(0, 112)],
                        aacc.at[pl.ds(sid * 624 + 512, 112)])

    @pl.when(sid == 15)
    def _():
        for j in range(5):
            pltpu.sync_copy(rA, aacc.at[pl.ds(9360 + j * 128, 128)])

    plsc.subcore_barrier()

    cfs = (cfA, cfB, cfC)
    dcs = (dcA, dcB, dcC)
    rows = (rA, rB, rC)
    msem = (mA, mB, mC)
    csem = (cA, cB, cC)
    dsem = (dA, dB, dC)
    ssem = (sA, sB, sC)
    qsem = (qA, qB, qC)

    def scale(rr, cfc, ngrp):
        def scale_grp(jg, carry2):
            cf16 = cfc[pl.ds(jg * 16, 16)]
            for j in range(16):
                sv = cf16[j]
                r = jg * 16 + j
                for kk in range(8):
                    sl = pl.ds(kk * 16, 16)
                    rr[r, sl] = rr[r, sl] * sv
            return carry2
        lax.fori_loop(0, ngrp, scale_grp, 0)

    def body(t, carry):
        o0 = t * 3 * BSZ
        mh = []
        ch = []
        dh = []
        for s in range(3):
            o = pl.ds(base + o0 + s * BSZ, BSZ)
            mh.append(pltpu.async_copy(m_hbm.at[o], rows[s], msem[s]))
            ch.append(pltpu.async_copy(cf_hbm.at[o], cfs[s], csem[s]))
            dh.append(pltpu.async_copy(dst_hbm.at[o], dcs[s], dsem[s]))
        sh = []
        qh = []
        for s in range(3):
            mh[s].wait()
            ch[s].wait()
            dh[s].wait()
            scale(rows[s], cfs[s], BSZ // 16)
            sh.append(pltpu.async_copy(rows[s], aacc.at[dcs[s]], ssem[s],
                                       add=True))
            qh.append(pltpu.async_copy(cfs[s], sacc.at[dcs[s]], qsem[s],
                                       add=True))
        for s in range(3):
            sh[s].wait()
            qh[s].wait()
        return carry

    lax.fori_loop(0, (NCH - 1) // 3, body, 0)
    off = (NCH - 1) * BSZ
    pltpu.sync_copy(m_hbm.at[pl.ds(base + off, TAIL)], rA.at[pl.ds(0, TAIL)])
    pltpu.sync_copy(cf_hbm.at[pl.ds(base + off, TAIL)],
                    cfA.at[pl.ds(0, TAIL)])
    pltpu.sync_copy(dst_hbm.at[pl.ds(base + off, TAIL)],
                    dcA.at[pl.ds(0, TAIL)])
    scale(rA, cfA, TAIL // 16)
    pltpu.sync_copy(rA.at[pl.ds(0, TAIL)],
                    aacc.at[dcA.at[pl.ds(0, TAIL)]], add=True)
    pltpu.sync_copy(cfA.at[pl.ds(0, TAIL)],
                    sacc.at[dcA.at[pl.ds(0, TAIL)]], add=True)

    plsc.subcore_barrier()

    @pl.when(sid < 15)
    def _():
        for j in range(4):
            pltpu.sync_copy(aacc.at[pl.ds(sid * 624 + j * 128, 128)],
                            agg_out.at[cid, pl.ds(sid * 624 + j * 128, 128)])
        pltpu.sync_copy(aacc.at[pl.ds(sid * 624 + 512, 112)],
                        agg_out.at[cid, pl.ds(sid * 624 + 512, 112)])
        pltpu.sync_copy(sacc.at[pl.ds(sid * 640, 640)],
                        sparts_out.at[cid, pl.ds(sid * 640, 640)])

    @pl.when(sid == 15)
    def _():
        for j in range(5):
            pltpu.sync_copy(aacc.at[pl.ds(9360 + j * 128, 128)],
                            agg_out.at[cid, pl.ds(9360 + j * 128, 128)])
        pltpu.sync_copy(sacc.at[pl.ds(9600, 512)],
                        sparts_out.at[cid, pl.ds(9600, 512)])


@functools.lru_cache(maxsize=None)
def _sc_kernels():
    mesh = _mesh()
    cp = pltpu.CompilerParams(needs_layout_passes=False)
    gate_gather = pl.kernel(
        _sc_gate_gather_body,
        out_type=(jax.ShapeDtypeStruct((E,), jnp.int32),
                  jax.ShapeDtypeStruct((E,), jnp.int32),
                  jax.ShapeDtypeStruct((E,), jnp.float32),
                  jax.ShapeDtypeStruct((E,), jnp.float32)),
        mesh=mesh,
        compiler_params=cp,
        scratch_types=[
            pltpu.VMEM((N,), jnp.int32),
            pltpu.VMEM((N,), jnp.float32),
            pltpu.VMEM((EPW,), jnp.int32),
            pltpu.VMEM((EPW,), jnp.int32),
            pltpu.VMEM((EPW,), jnp.int32),
            pltpu.VMEM((EPW,), jnp.int32),
            pltpu.VMEM((EPW,), jnp.float32),
            pltpu.VMEM((EPW,), jnp.float32),
        ],
    )
    row_gather = pl.kernel(
        _sc_row_gather_body,
        out_type=jax.ShapeDtypeStruct((E, HID), jnp.float32),
        mesh=mesh,
        compiler_params=cp,
        scratch_types=(
            [pltpu.VMEM((EPW,), jnp.int32)]
            + [pltpu.VMEM((BSZ, HID), jnp.float32)] * 4
            + [pltpu.SemaphoreType.DMA] * 8
        ),
    )

    def make_denom(do_cnt):
        return pl.kernel(
            functools.partial(_sc_denom_body, do_cnt),
            out_type=(jax.ShapeDtypeStruct((E,), jnp.float32),
                      jax.ShapeDtypeStruct((NC, NPAD), jnp.float32),
                      jax.ShapeDtypeStruct((NC, NPAD), jnp.float32)),
            mesh=mesh,
            compiler_params=cp,
            scratch_types=[
                pltpu.VMEM((EPW,), jnp.float32),
                pltpu.VMEM((EPW,), jnp.float32),
                pltpu.VMEM((EPW,), jnp.float32),
                pltpu.VMEM((EPW,), jnp.float32),
                pltpu.VMEM((EPW,), jnp.int32),
                pltpu.VMEM((BSZ,), jnp.float32),
                pltpu.VMEM((16,), jnp.float32),
                pltpu.VMEM((640,), jnp.float32),
                pltpu.VMEM_SHARED((NPAD,), jnp.float32),
                pltpu.VMEM_SHARED((NPAD,), jnp.float32),
            ],
        )

    coeff = pl.kernel(
        _sc_coeff_body,
        out_type=jax.ShapeDtypeStruct((E,), jnp.float32),
        mesh=mesh,
        compiler_params=cp,
        scratch_types=[
            pltpu.VMEM((NPAD,), jnp.float32),
            pltpu.VMEM((NPAD,), jnp.float32),
            pltpu.VMEM((5120,), jnp.float32),
            pltpu.VMEM((EPW,), jnp.float32),
            pltpu.VMEM((EPW,), jnp.float32),
            pltpu.VMEM((EPW,), jnp.int32),
        ],
    )
    aggregate = pl.kernel(
        _sc_aggregate_body,
        out_type=(jax.ShapeDtypeStruct((NC, N, HID), jnp.float32),
                  jax.ShapeDtypeStruct((NC, NPAD), jnp.float32)),
        mesh=mesh,
        compiler_params=cp,
        scratch_types=(
            [pltpu.VMEM((BSZ,), jnp.float32)] * 3
            + [pltpu.VMEM((BSZ,), jnp.int32)] * 3
            + [pltpu.VMEM((BSZ, HID), jnp.float32)] * 3
            + [pltpu.VMEM_SHARED((N, HID), jnp.float32),
               pltpu.VMEM_SHARED((NPAD,), jnp.float32)]
            + [pltpu.SemaphoreType.DMA] * 15
        ),
    )
    return gate_gather, row_gather, make_denom(True), make_denom(False), coeff, aggregate


_NBA = 1000
_BE = 2000


def _prelude(x, nt3, wx, tn, b, wa):
    return pl.pallas_call(
        _prelude_body,
        grid=(N // _NBA,),
        in_specs=[
            pl.BlockSpec((_NBA, HID), lambda i: (i, 0)),
            pl.BlockSpec((1, 1, _NBA), lambda i: (i, 0, 0)),
            pl.BlockSpec((HID, HID), lambda i: (0, 0)),
            pl.BlockSpec((NTYPE, HID), lambda i: (0, 0)),
            pl.BlockSpec((1, HID), lambda i: (0, 0)),
            pl.BlockSpec((HID, HID), lambda i: (0, 0)),
        ],
        out_specs=pl.BlockSpec((_NBA, HID), lambda i: (i, 0)),
        out_shape=jax.ShapeDtypeStruct((N, HID), jnp.float32),
    )(x, nt3, wx, tn, b, wa)


def _gate(nts3, ntd3, lms3, lmd3, ts, td, us, ud, b1, w2t, b2):
    gbl = E // _BE
    return pl.pallas_call(
        _gate_body,
        grid=(gbl,),
        in_specs=[
            pl.BlockSpec((1, 1, _BE), lambda i: (i, 0, 0)),
            pl.BlockSpec((1, 1, _BE), lambda i: (i, 0, 0)),
            pl.BlockSpec((1, 1, _BE), lambda i: (i, 0, 0)),
            pl.BlockSpec((1, 1, _BE), lambda i: (i, 0, 0)),
            pl.BlockSpec((NTYPE, NTYPE), lambda i: (0, 0)),
            pl.BlockSpec((NTYPE, NTYPE), lambda i: (0, 0)),
            pl.BlockSpec((1, NTYPE), lambda i: (0, 0)),
            pl.BlockSpec((1, NTYPE), lambda i: (0, 0)),
            pl.BlockSpec((1, NTYPE), lambda i: (0, 0)),
            pl.BlockSpec((1, NTYPE), lambda i: (0, 0)),
            pl.BlockSpec((1, 1), lambda i: (0, 0)),
        ],
        out_specs=pl.BlockSpec((1, 1, _BE), lambda i: (i, 0, 0)),
        out_shape=jax.ShapeDtypeStruct((gbl, 1, _BE), jnp.float32),
    )(nts3, ntd3, lms3, lmd3, ts, td, us, ud, b1, w2t, b2)


def _edge_mlp(mpre, et3, qb, wtil, c):
    gbl = E // _BE
    return pl.pallas_call(
        _edge_mlp_body,
        grid=(gbl,),
        in_specs=[
            pl.BlockSpec((_BE, HID), lambda i: (i, 0)),
            pl.BlockSpec((1, 1, _BE), lambda i: (i, 0, 0)),
            pl.BlockSpec((ETYPE, HID), lambda i: (0, 0)),
            pl.BlockSpec((1, HID), lambda i: (0, 0)),
            pl.BlockSpec((1, 1), lambda i: (0, 0)),
        ],
        out_specs=[
            pl.BlockSpec((_BE, HID), lambda i: (i, 0)),
            pl.BlockSpec((1, 1, _BE), lambda i: (i, 0, 0)),
            pl.BlockSpec((8, 128), lambda i: (0, 0)),
        ],
        out_shape=[
            jax.ShapeDtypeStruct((E, HID), jnp.float32),
            jax.ShapeDtypeStruct((gbl, 1, _BE), jnp.float32),
            jax.ShapeDtypeStruct((8, 128), jnp.float32),
        ],
        scratch_shapes=[pltpu.SMEM((1, 1), jnp.float32)],
    )(mpre, et3, qb, wtil, c)


def _combine(parts, sp3, w2, b2, wa):
    return pl.pallas_call(
        _combine_body,
        grid=(N // _NBA,),
        in_specs=[
            pl.BlockSpec((NC, _NBA, HID), lambda i: (0, i, 0)),
            pl.BlockSpec((1, NC, _NBA), lambda i: (i, 0, 0)),
            pl.BlockSpec((HID, HID), lambda i: (0, 0)),
            pl.BlockSpec((1, HID), lambda i: (0, 0)),
            pl.BlockSpec((HID, HID), lambda i: (0, 0)),
        ],
        out_specs=pl.BlockSpec((_NBA, HID), lambda i: (i, 0)),
        out_shape=jax.ShapeDtypeStruct((N, HID), jnp.float32),
    )(parts, sp3, w2, b2, wa)


def _final(parts, sp3, w2, b2, fcw, fcb):
    return pl.pallas_call(
        _final_body,
        grid=(N // _NBA,),
        in_specs=[
            pl.BlockSpec((NC, _NBA, HID), lambda i: (0, i, 0)),
            pl.BlockSpec((1, NC, _NBA), lambda i: (i, 0, 0)),
            pl.BlockSpec((HID, HID), lambda i: (0, 0)),
            pl.BlockSpec((1, HID), lambda i: (0, 0)),
            pl.BlockSpec((HID, 16), lambda i: (0, 0)),
            pl.BlockSpec((1, 16), lambda i: (0, 0)),
        ],
        out_specs=pl.BlockSpec((1, 16), lambda i: (0, 0)),
        out_shape=jax.ShapeDtypeStruct((1, 16), jnp.float32),
        scratch_shapes=[pltpu.VMEM((1, HID), jnp.float32)],
    )(parts, sp3, w2, b2, fcw, fcb)


def kernel(x, node_type, edge_index, edge_type, landmark_mask, nt_table,
           in_proj_W, in_proj_b, gate_W1, gate_b1, gate_W2, gate_b2,
           c1_edge_emb, c1_msg_W1, c1_msg_b1, c1_msg_W2, c1_msg_b2,
           c1_att_W, c1_att_b, c2_edge_emb, c2_msg_W1, c2_msg_b1,
           c2_msg_W2, c2_msg_b2, c2_att_W, c2_att_b, fc_W, fc_b):
    f32 = jnp.float32
    node_type = node_type.astype(jnp.int32)
    edge_type = edge_type.astype(jnp.int32)
    src = edge_index[0].astype(jnp.int32)
    dst = edge_index[1].astype(jnp.int32)

    wx = in_proj_W[:HID]
    tn = jnp.dot(nt_table, in_proj_W[HID:], preferred_element_type=f32)
    b_in = in_proj_b[None, :]
    ts = jnp.dot(nt_table, gate_W1[0:8], preferred_element_type=f32)
    td = jnp.dot(nt_table, gate_W1[8:16], preferred_element_type=f32)
    us = gate_W1[16][None, :]
    ud = gate_W1[17][None, :]
    gb1 = gate_b1[None, :]
    w2t = gate_W2.T
    gb2 = gate_b2.reshape(1, 1)

    nt3 = node_type.reshape(N // _NBA, 1, _NBA)
    et3 = edge_type.reshape(E // _BE, 1, _BE)

    (_sc_gate_gather, _sc_row_gather, _sc_denom_cnt, _sc_denom_nocnt,
     _sc_coeff, _sc_aggregate) = _sc_kernels()

    nts, ntd, lms, lmd = _sc_gate_gather(node_type, landmark_mask, src, dst)
    gbl = E // _BE
    g = _gate(nts.reshape(gbl, 1, _BE), ntd.reshape(gbl, 1, _BE),
              lms.reshape(gbl, 1, _BE), lmd.reshape(gbl, 1, _BE),
              ts, td, us, ud, gb1, w2t, gb2).reshape(E)

    wa1 = c1_msg_W1[:HID]
    p1 = _prelude(x, nt3, wx, tn, b_in, wa1)

    def conv(p, edge_emb, msg_W1, msg_b1, msg_W2, msg_b2, att_W, att_b,
             do_cnt, cparts_prev):
        qb = jnp.dot(edge_emb, msg_W1[HID:], preferred_element_type=f32) \
            + msg_b1[None, :]
        wtil = jnp.dot(msg_W2, att_W, preferred_element_type=f32).T
        cconst = (jnp.dot(msg_b2[None, :], att_W, preferred_element_type=f32)
                  + att_b).reshape(1, 1)
        mpre = _sc_row_gather(p, src)
        r, a2, gmax8 = _edge_mlp(mpre, et3, qb, wtil, cconst)
        gmax16 = gmax8.reshape(-1)[:16]
        if do_cnt:
            eg, dparts, cparts = _sc_denom_cnt(a2.reshape(E), g, gmax16, dst)
        else:
            eg, dparts, _ = _sc_denom_nocnt(a2.reshape(E), g, gmax16, dst)
            cparts = cparts_prev
        cf = _sc_coeff(eg, dst, dparts, cparts)
        agg, sparts = _sc_aggregate(r, cf, dst)
        sp3 = sparts[:, :N].reshape(NC, N // _NBA, _NBA).transpose(1, 0, 2)
        return agg, sp3, cparts

    agg1, sp31, cparts = conv(p1, c1_edge_emb, c1_msg_W1, c1_msg_b1,
                              c1_msg_W2, c1_msg_b2, c1_att_W, c1_att_b,
                              True, None)
    p2 = _combine(agg1, sp31, c1_msg_W2, c1_msg_b2[None, :], c2_msg_W1[:HID])
    agg2, sp32, _ = conv(p2, c2_edge_emb, c2_msg_W1, c2_msg_b1, c2_msg_W2,
                         c2_msg_b2, c2_att_W, c2_att_b, False, cparts)
    return _final(agg2, sp32, c2_msg_W2, c2_msg_b2[None, :], fc_W,
                  fc_b.reshape(1, 16))


# speedup vs baseline: 11.5206x; 1.0004x over previous
import functools

import jax
import jax.numpy as jnp
from jax import lax
from jax.experimental import pallas as pl
from jax.experimental.pallas import tpu as pltpu
from jax.experimental.pallas import tpu_sc as plsc

N = 10000
E = 320000
HID = 128
NTYPE = 32
ETYPE = 16

NC = 2
NS = 16
NW = NC * NS
EPW = E // NW
BSZ = 128
NPAD = 10112
NEG_BIG = -3.0e38

@functools.lru_cache(maxsize=None)
def _mesh():
    return plsc.VectorSubcoreMesh(core_axis_name="c", subcore_axis_name="s",
                                  num_cores=NC, num_subcores=NS)


def _wid():
    return lax.axis_index("s") * NC + lax.axis_index("c")


def _prelude_body(x_ref, nt3_ref, wx_ref, tn_ref, b_ref, wa_ref, p_ref):
    nt = nt3_ref[0, 0, :]
    oh = (nt[:, None] == lax.broadcasted_iota(jnp.int32, (nt.shape[0], NTYPE), 1)
          ).astype(jnp.float32)
    h = jnp.dot(x_ref[...], wx_ref[...], preferred_element_type=jnp.float32)
    h = h + jnp.dot(oh, tn_ref[...], preferred_element_type=jnp.float32)
    h = jax.nn.relu(h + b_ref[...])
    p_ref[...] = jnp.dot(h, wa_ref[...], preferred_element_type=jnp.float32)


def _gate_body(nts_ref, ntd_ref, lms_ref, lmd_ref, ts_ref, td_ref, us_ref,
               ud_ref, b1_ref, w2t_ref, b2_ref, g_ref):
    nts = nts_ref[0, 0, :]
    ntd = ntd_ref[0, 0, :]
    be = nts.shape[0]
    iota = lax.broadcasted_iota(jnp.int32, (be, NTYPE), 1)
    ohs = (nts[:, None] == iota).astype(jnp.float32)
    ohd = (ntd[:, None] == iota).astype(jnp.float32)
    hid = jnp.dot(ohs, ts_ref[...], preferred_element_type=jnp.float32)
    hid = hid + jnp.dot(ohd, td_ref[...], preferred_element_type=jnp.float32)
    hid = hid + lms_ref[0, 0, :][:, None] * us_ref[...]
    hid = hid + lmd_ref[0, 0, :][:, None] * ud_ref[...]
    hid = jax.nn.relu(hid + b1_ref[...])
    grow = lax.dot_general(w2t_ref[...], hid, (((1,), (1,)), ((), ())),
                           preferred_element_type=jnp.float32)
    g_ref[...] = jax.nn.sigmoid(grow + b2_ref[...])[:, None, :]


def _edge_mlp_body(mpre_ref, et3_ref, qb_ref, wtil_ref, c_ref,
                   m_ref, a_ref, gmax_ref, mx_sc):
    i = pl.program_id(0)
    ni = pl.num_programs(0)
    et = et3_ref[0, 0, :]
    oh = (et[:, None] == lax.broadcasted_iota(jnp.int32, (et.shape[0], ETYPE), 1)
          ).astype(jnp.float32)
    m1 = mpre_ref[...] + jnp.dot(oh, qb_ref[...], preferred_element_type=jnp.float32)
    r = jax.nn.relu(m1)
    m_ref[...] = r
    arow = lax.dot_general(wtil_ref[...], r, (((1,), (1,)), ((), ())),
                           preferred_element_type=jnp.float32)
    arow = arow + c_ref[...]
    a_ref[...] = arow[:, None, :]
    bmax = jnp.max(arow)
    prev = jnp.where(i == 0, NEG_BIG, mx_sc[0, 0])
    mx_sc[0, 0] = jnp.maximum(prev, bmax)

    @pl.when(i == ni - 1)
    def _():
        gmax_ref[...] = jnp.full((8, 128), mx_sc[0, 0], dtype=jnp.float32)


def _node_h(parts_ref, sp3_ref, w2_ref, b2_ref):
    s = sp3_ref[0, 0, :] + sp3_ref[0, 1, :]
    h = jnp.dot(parts_ref[0] + parts_ref[1], w2_ref[...],
                preferred_element_type=jnp.float32)
    return jax.nn.relu(h + s[:, None] * b2_ref[...])


def _combine_body(parts_ref, sp3_ref, w2_ref, b2_ref, wa_ref, p_ref):
    h = _node_h(parts_ref, sp3_ref, w2_ref, b2_ref)
    p_ref[...] = jnp.dot(h, wa_ref[...], preferred_element_type=jnp.float32)


def _final_body(parts_ref, sp3_ref, w2_ref, b2_ref, fcw_ref, fcb_ref,
                out_ref, acc):
    i = pl.program_id(0)
    ni = pl.num_programs(0)
    h = _node_h(parts_ref, sp3_ref, w2_ref, b2_ref)
    s = jnp.sum(h, axis=0, keepdims=True)

    @pl.when(i == 0)
    def _():
        acc[...] = s

    @pl.when(i != 0)
    def _():
        acc[...] = acc[...] + s

    @pl.when(i == ni - 1)
    def _():
        out_ref[...] = (jnp.dot(acc[...] * (1.0 / N), fcw_ref[...],
                                preferred_element_type=jnp.float32)
                        + fcb_ref[...])


def _sc_gate_gather_body(nt_hbm, lm_hbm, src_hbm, dst_hbm,
                         nts_out, ntd_out, lms_out, lmd_out,
                         ntb, lmb, srcb, dstb, ntsb, ntdb, lmsb, lmdb):
    wid = _wid()
    base = wid * EPW
    pltpu.sync_copy(nt_hbm, ntb)
    pltpu.sync_copy(lm_hbm, lmb)
    pltpu.sync_copy(src_hbm.at[pl.ds(base, EPW)], srcb)
    pltpu.sync_copy(dst_hbm.at[pl.ds(base, EPW)], dstb)

    def body(i, carry):
        sl = pl.ds(i * 16, 16)
        si = srcb[sl]
        di = dstb[sl]
        ntsb[sl] = plsc.load_gather(ntb, [si])
        lmsb[sl] = plsc.load_gather(lmb, [si])
        ntdb[sl] = plsc.load_gather(ntb, [di])
        lmdb[sl] = plsc.load_gather(lmb, [di])
        return carry

    lax.fori_loop(0, EPW // 16, body, 0)
    pltpu.sync_copy(ntsb, nts_out.at[pl.ds(base, EPW)])
    pltpu.sync_copy(ntdb, ntd_out.at[pl.ds(base, EPW)])
    pltpu.sync_copy(lmsb, lms_out.at[pl.ds(base, EPW)])
    pltpu.sync_copy(lmdb, lmd_out.at[pl.ds(base, EPW)])


NCH = -(-EPW // BSZ)
TAIL = EPW - (NCH - 1) * BSZ


def _sc_row_gather_body(p_hbm, src_hbm, out_hbm, srcb,
                        rA, rB, rC, rD, gA, gB, gC, gD, oA, oB, oC, oD):
    wid = _wid()
    base = wid * EPW
    pltpu.sync_copy(src_hbm.at[pl.ds(base, EPW)], srcb)
    bufs = (rA, rB, rC, rD)
    gsem = (gA, gB, gC, gD)
    osem = (oA, oB, oC, oD)

    def body(t, carry):
        o0 = t * 4 * BSZ
        gh = [pltpu.async_copy(
            p_hbm.at[srcb.at[pl.ds(o0 + s * BSZ, BSZ)]], bufs[s], gsem[s])
            for s in range(4)]
        oh = []
        for s in range(4):
            gh[s].wait()
            oh.append(pltpu.async_copy(
                bufs[s], out_hbm.at[pl.ds(base + o0 + s * BSZ, BSZ)],
                osem[s]))
        for s in range(4):
            oh[s].wait()
        return carry

    nquad = (NCH - 1) // 4
    lax.fori_loop(0, nquad, body, 0)
    o0 = nquad * 4 * BSZ
    gh = [pltpu.async_copy(
        p_hbm.at[srcb.at[pl.ds(o0 + s * BSZ, BSZ)]], bufs[s], gsem[s])
        for s in range((NCH - 1) % 4)]
    oh = []
    for s in range((NCH - 1) % 4):
        gh[s].wait()
        oh.append(pltpu.async_copy(
            bufs[s], out_hbm.at[pl.ds(base + o0 + s * BSZ, BSZ)], osem[s]))
    for h in oh:
        h.wait()
    off = (NCH - 1) * BSZ
    pltpu.async_copy(p_hbm.at[srcb.at[pl.ds(off, TAIL)]],
                     rA.at[pl.ds(0, TAIL)], gA).wait()
    pltpu.sync_copy(rA.at[pl.ds(0, TAIL)],
                    out_hbm.at[pl.ds(base + off, TAIL)])


def _sc_denom_body(do_cnt, a_hbm, g_hbm, gmax_hbm, dst_hbm,
                   eg_out, dpart_out, cpart_out,
                   ab, gb, egb, eb, dstb, onesb, gmaxb, zb, dacc, cacc):
    cid = lax.axis_index("c")
    sid = lax.axis_index("s")
    wid = sid * NC + cid
    base = wid * EPW

    def zbody(i, carry):
        zb[pl.ds(i * 16, 16)] = jnp.zeros((16,), jnp.float32)
        return carry
    lax.fori_loop(0, 40, zbody, 0)

    @pl.when(sid < 15)
    def _():
        pltpu.sync_copy(zb, dacc.at[pl.ds(sid * 640, 640)])
        if do_cnt:
            pltpu.sync_copy(zb, cacc.at[pl.ds(sid * 640, 640)])

    @pl.when(sid == 15)
    def _():
        pltpu.sync_copy(zb.at[pl.ds(0, 512)], dacc.at[pl.ds(9600, 512)])
        if do_cnt:
            pltpu.sync_copy(zb.at[pl.ds(0, 512)], cacc.at[pl.ds(9600, 512)])

    pltpu.sync_copy(gmax_hbm, gmaxb)
    pltpu.sync_copy(a_hbm.at[pl.ds(base, EPW)], ab)
    pltpu.sync_copy(g_hbm.at[pl.ds(base, EPW)], gb)
    pltpu.sync_copy(dst_hbm.at[pl.ds(base, EPW)], dstb)
    if do_cnt:
        def obody(i, carry):
            onesb[pl.ds(i * 16, 16)] = jnp.ones((16,), jnp.float32)
            return carry
        lax.fori_loop(0, BSZ // 16, obody, 0)

    gmax = gmaxb[...]

    def ebody(i, carry2):
        sl = pl.ds(i * 16, 16)
        ev = jnp.exp(ab[sl] - gmax)
        eb[sl] = ev
        egb[sl] = ev * gb[sl]
        return carry2
    lax.fori_loop(0, EPW // 16, ebody, 0)

    plsc.subcore_barrier()

    pltpu.sync_copy(egb, eg_out.at[pl.ds(base, EPW)])

    def sbody(i, carry):
        sl = pl.ds(i * BSZ, BSZ)
        idx = dstb.at[sl]
        pltpu.sync_copy(eb.at[sl], dacc.at[idx], add=True)
        if do_cnt:
            pltpu.sync_copy(onesb, cacc.at[idx], add=True)
        return carry
    lax.fori_loop(0, NCH - 1, sbody, 0)
    tsl = pl.ds((NCH - 1) * BSZ, TAIL)
    pltpu.sync_copy(eb.at[tsl], dacc.at[dstb.at[tsl]], add=True)
    if do_cnt:
        pltpu.sync_copy(onesb.at[pl.ds(0, TAIL)], cacc.at[dstb.at[tsl]],
                        add=True)

    plsc.subcore_barrier()

    @pl.when(sid < 15)
    def _():
        pltpu.sync_copy(dacc.at[pl.ds(sid * 640, 640)],
                        dpart_out.at[cid, pl.ds(sid * 640, 640)])
        if do_cnt:
            pltpu.sync_copy(cacc.at[pl.ds(sid * 640, 640)],
                            cpart_out.at[cid, pl.ds(sid * 640, 640)])

    @pl.when(sid == 15)
    def _():
        pltpu.sync_copy(dacc.at[pl.ds(9600, 512)],
                        dpart_out.at[cid, pl.ds(9600, 512)])
        if do_cnt:
            pltpu.sync_copy(cacc.at[pl.ds(9600, 512)],
                            cpart_out.at[cid, pl.ds(9600, 512)])


def _sc_coeff_body(eg_hbm, dst_hbm, dpart_hbm, cpart_hbm, cf_out,
                   denb, cntb, tmp, egb, cfb, dstf):
    wid = _wid()
    base = wid * EPW

    pltpu.sync_copy(dpart_hbm.at[0], denb)
    pltpu.sync_copy(cpart_hbm.at[0], cntb)
    for part, accb in ((dpart_hbm, denb), (cpart_hbm, cntb)):
        for off, ln in ((0, 5120), (5120, 4992)):
            pltpu.sync_copy(part.at[1, pl.ds(off, ln)], tmp.at[pl.ds(0, ln)])

            def abody(i, carry2, off=off, accb=accb):
                s2 = pl.ds(off + i * 16, 16)
                accb[s2] = accb[s2] + tmp[pl.ds(i * 16, 16)]
                return carry2
            lax.fori_loop(0, ln // 16, abody, 0)

    pltpu.sync_copy(eg_hbm.at[pl.ds(base, EPW)], egb)
    pltpu.sync_copy(dst_hbm.at[pl.ds(base, EPW)], dstf)

    def cfbody(i, carry):
        sl = pl.ds(i * 16, 16)
        idx = dstf[sl]
        d16 = plsc.load_gather(denb, [idx])
        c16 = plsc.load_gather(cntb, [idx])
        cfb[sl] = egb[sl] / (d16 + 1e-16) / jnp.maximum(c16, 1.0)
        return carry
    lax.fori_loop(0, EPW // 16, cfbody, 0)
    pltpu.sync_copy(cfb, cf_out.at[pl.ds(base, EPW)])


def _sc_aggregate_body(m_hbm, cf_hbm, dst_hbm, agg_out, sparts_out,
                       cfA, cfB, cfC, dcA, dcB, dcC, rA, rB, rC, aacc, sacc,
                       mA, mB, mC, cA, cB, cC, dA, dB, dC,
                       sA, sB, sC, qA, qB, qC):
    cid = lax.axis_index("c")
    sid = lax.axis_index("s")
    wid = sid * NC + cid
    base = wid * EPW

    def zcf(i, carry):
        cfA[pl.ds(i * 16, 16)] = jnp.zeros((16,), jnp.float32)
        return carry
    lax.fori_loop(0, BSZ // 16, zcf, 0)

    @pl.when(sid < 15)
    def _():
        for j in range(5):
            pltpu.sync_copy(cfA, sacc.at[pl.ds(sid * 640 + j * 128, 128)])

    @pl.when(sid == 15)
    def _():
        for j in range(4):
            pltpu.sync_copy(cfA, sacc.at[pl.ds(9600 + j * 128, 128)])

    def zrows(i, carry):
        r = i // 8
        k = i % 8
        rA[r, pl.ds(k * 16, 16)] = jnp.zeros((16,), jnp.float32)
        return carry
    lax.fori_loop(0, BSZ * 8, zrows, 0)

    @pl.when(sid < 15)
    def _():
        for j in range(4):
            pltpu.sync_copy(rA, aacc.at[pl.ds(sid * 624 + j * 128, 128)])
        pltpu.sync_copy(rA.at[pl.ds(0, 112)],
                        aacc.at[pl.ds(sid * 624 + 512, 112)])

    @pl.when(sid == 15)
    def _():
        for j in range(5):
            pltpu.sync_copy(rA, aacc.at[pl.ds(9360 + j * 128, 128)])

    plsc.subcore_barrier()

    cfs = (cfA, cfB, cfC)
    dcs = (dcA, dcB, dcC)
    rows = (rA, rB, rC)
    msem = (mA, mB, mC)
    csem = (cA, cB, cC)
    dsem = (dA, dB, dC)
    ssem = (sA, sB, sC)
    qsem = (qA, qB, qC)

    def scale(rr, cfc, ngrp):
        def scale_grp(jg, carry2):
            cf16 = cfc[pl.ds(jg * 16, 16)]
            for j in range(16):
                sv = cf16[j]
                r = jg * 16 + j
                for kk in range(8):
                    sl = pl.ds(kk * 16, 16)
                    rr[r, sl] = rr[r, sl] * sv
            return carry2
        lax.fori_loop(0, ngrp, scale_grp, 0)

    def body(t, carry):
        o0 = t * 3 * BSZ
        mh = []
        ch = []
        dh = []
        for s in range(3):
            o = pl.ds(base + o0 + s * BSZ, BSZ)
            mh.append(pltpu.async_copy(m_hbm.at[o], rows[s], msem[s]))
            ch.append(pltpu.async_copy(cf_hbm.at[o], cfs[s], csem[s]))
            dh.append(pltpu.async_copy(dst_hbm.at[o], dcs[s], dsem[s]))
        sh = []
        qh = []
        for s in range(3):
            mh[s].wait()
            ch[s].wait()
            dh[s].wait()
            scale(rows[s], cfs[s], BSZ // 16)
            sh.append(pltpu.async_copy(rows[s], aacc.at[dcs[s]], ssem[s],
                                       add=True))
            qh.append(pltpu.async_copy(cfs[s], sacc.at[dcs[s]], qsem[s],
                                       add=True))
        for s in range(3):
            sh[s].wait()
            qh[s].wait()
        return carry

    lax.fori_loop(0, (NCH - 1) // 3, body, 0)
    off = (NCH - 1) * BSZ
    pltpu.sync_copy(m_hbm.at[pl.ds(base + off, TAIL)], rA.at[pl.ds(0, TAIL)])
    pltpu.sync_copy(cf_hbm.at[pl.ds(base + off, TAIL)],
                    cfA.at[pl.ds(0, TAIL)])
    pltpu.sync_copy(dst_hbm.at[pl.ds(base + off, TAIL)],
                    dcA.at[pl.ds(0, TAIL)])
    scale(rA, cfA, TAIL // 16)
    pltpu.sync_copy(rA.at[pl.ds(0, TAIL)],
                    aacc.at[dcA.at[pl.ds(0, TAIL)]], add=True)
    pltpu.sync_copy(cfA.at[pl.ds(0, TAIL)],
                    sacc.at[dcA.at[pl.ds(0, TAIL)]], add=True)

    plsc.subcore_barrier()

    @pl.when(sid < 15)
    def _():
        for j in range(4):
            pltpu.sync_copy(aacc.at[pl.ds(sid * 624 + j * 128, 128)],
                            agg_out.at[cid, pl.ds(sid * 624 + j * 128, 128)])
        pltpu.sync_copy(aacc.at[pl.ds(sid * 624 + 512, 112)],
                        agg_out.at[cid, pl.ds(sid * 624 + 512, 112)])
        pltpu.sync_copy(sacc.at[pl.ds(sid * 640, 640)],
                        sparts_out.at[cid, pl.ds(sid * 640, 640)])

    @pl.when(sid == 15)
    def _():
        for j in range(5):
            pltpu.sync_copy(aacc.at[pl.ds(9360 + j * 128, 128)],
                            agg_out.at[cid, pl.ds(9360 + j * 128, 128)])
        pltpu.sync_copy(sacc.at[pl.ds(9600, 512)],
                        sparts_out.at[cid, pl.ds(9600, 512)])


@functools.lru_cache(maxsize=None)
def _sc_kernels():
    mesh = _mesh()
    cp = pltpu.CompilerParams(needs_layout_passes=False)
    gate_gather = pl.kernel(
        _sc_gate_gather_body,
        out_type=(jax.ShapeDtypeStruct((E,), jnp.int32),
                  jax.ShapeDtypeStruct((E,), jnp.int32),
                  jax.ShapeDtypeStruct((E,), jnp.float32),
                  jax.ShapeDtypeStruct((E,), jnp.float32)),
        mesh=mesh,
        compiler_params=cp,
        scratch_types=[
            pltpu.VMEM((N,), jnp.int32),
            pltpu.VMEM((N,), jnp.float32),
            pltpu.VMEM((EPW,), jnp.int32),
            pltpu.VMEM((EPW,), jnp.int32),
            pltpu.VMEM((EPW,), jnp.int32),
            pltpu.VMEM((EPW,), jnp.int32),
            pltpu.VMEM((EPW,), jnp.float32),
            pltpu.VMEM((EPW,), jnp.float32),
        ],
    )
    row_gather = pl.kernel(
        _sc_row_gather_body,
        out_type=jax.ShapeDtypeStruct((E, HID), jnp.float32),
        mesh=mesh,
        compiler_params=cp,
        scratch_types=(
            [pltpu.VMEM((EPW,), jnp.int32)]
            + [pltpu.VMEM((BSZ, HID), jnp.float32)] * 4
            + [pltpu.SemaphoreType.DMA] * 8
        ),
    )

    def make_denom(do_cnt):
        return pl.kernel(
            functools.partial(_sc_denom_body, do_cnt),
            out_type=(jax.ShapeDtypeStruct((E,), jnp.float32),
                      jax.ShapeDtypeStruct((NC, NPAD), jnp.float32),
                      jax.ShapeDtypeStruct((NC, NPAD), jnp.float32)),
            mesh=mesh,
            compiler_params=cp,
            scratch_types=[
                pltpu.VMEM((EPW,), jnp.float32),
                pltpu.VMEM((EPW,), jnp.float32),
                pltpu.VMEM((EPW,), jnp.float32),
                pltpu.VMEM((EPW,), jnp.float32),
                pltpu.VMEM((EPW,), jnp.int32),
                pltpu.VMEM((BSZ,), jnp.float32),
                pltpu.VMEM((16,), jnp.float32),
                pltpu.VMEM((640,), jnp.float32),
                pltpu.VMEM_SHARED((NPAD,), jnp.float32),
                pltpu.VMEM_SHARED((NPAD,), jnp.float32),
            ],
        )

    coeff = pl.kernel(
        _sc_coeff_body,
        out_type=jax.ShapeDtypeStruct((E,), jnp.float32),
        mesh=mesh,
        compiler_params=cp,
        scratch_types=[
            pltpu.VMEM((NPAD,), jnp.float32),
            pltpu.VMEM((NPAD,), jnp.float32),
            pltpu.VMEM((5120,), jnp.float32),
            pltpu.VMEM((EPW,), jnp.float32),
            pltpu.VMEM((EPW,), jnp.float32),
            pltpu.VMEM((EPW,), jnp.int32),
        ],
    )
    aggregate = pl.kernel(
        _sc_aggregate_body,
        out_type=(jax.ShapeDtypeStruct((NC, N, HID), jnp.float32),
                  jax.ShapeDtypeStruct((NC, NPAD), jnp.float32)),
        mesh=mesh,
        compiler_params=cp,
        scratch_types=(
            [pltpu.VMEM((BSZ,), jnp.float32)] * 3
            + [pltpu.VMEM((BSZ,), jnp.int32)] * 3
            + [pltpu.VMEM((BSZ, HID), jnp.float32)] * 3
            + [pltpu.VMEM_SHARED((N, HID), jnp.float32),
               pltpu.VMEM_SHARED((NPAD,), jnp.float32)]
            + [pltpu.SemaphoreType.DMA] * 15
        ),
    )
    return gate_gather, row_gather, make_denom(True), make_denom(False), coeff, aggregate


_NBA = 1000
_BE = 2000


def _prelude(x, nt3, wx, tn, b, wa):
    return pl.pallas_call(
        _prelude_body,
        grid=(N // _NBA,),
        in_specs=[
            pl.BlockSpec((_NBA, HID), lambda i: (i, 0)),
            pl.BlockSpec((1, 1, _NBA), lambda i: (i, 0, 0)),
            pl.BlockSpec((HID, HID), lambda i: (0, 0)),
            pl.BlockSpec((NTYPE, HID), lambda i: (0, 0)),
            pl.BlockSpec((1, HID), lambda i: (0, 0)),
            pl.BlockSpec((HID, HID), lambda i: (0, 0)),
        ],
        out_specs=pl.BlockSpec((_NBA, HID), lambda i: (i, 0)),
        out_shape=jax.ShapeDtypeStruct((N, HID), jnp.float32),
    )(x, nt3, wx, tn, b, wa)


def _gate(nts3, ntd3, lms3, lmd3, ts, td, us, ud, b1, w2t, b2):
    gbl = E // _BE
    return pl.pallas_call(
        _gate_body,
        grid=(gbl,),
        in_specs=[
            pl.BlockSpec((1, 1, _BE), lambda i: (i, 0, 0)),
            pl.BlockSpec((1, 1, _BE), lambda i: (i, 0, 0)),
            pl.BlockSpec((1, 1, _BE), lambda i: (i, 0, 0)),
            pl.BlockSpec((1, 1, _BE), lambda i: (i, 0, 0)),
            pl.BlockSpec((NTYPE, NTYPE), lambda i: (0, 0)),
            pl.BlockSpec((NTYPE, NTYPE), lambda i: (0, 0)),
            pl.BlockSpec((1, NTYPE), lambda i: (0, 0)),
            pl.BlockSpec((1, NTYPE), lambda i: (0, 0)),
            pl.BlockSpec((1, NTYPE), lambda i: (0, 0)),
            pl.BlockSpec((1, NTYPE), lambda i: (0, 0)),
            pl.BlockSpec((1, 1), lambda i: (0, 0)),
        ],
        out_specs=pl.BlockSpec((1, 1, _BE), lambda i: (i, 0, 0)),
        out_shape=jax.ShapeDtypeStruct((gbl, 1, _BE), jnp.float32),
    )(nts3, ntd3, lms3, lmd3, ts, td, us, ud, b1, w2t, b2)


def _edge_mlp(mpre, et3, qb, wtil, c):
    gbl = E // _BE
    return pl.pallas_call(
        _edge_mlp_body,
        grid=(gbl,),
        in_specs=[
            pl.BlockSpec((_BE, HID), lambda i: (i, 0)),
            pl.BlockSpec((1, 1, _BE), lambda i: (i, 0, 0)),
            pl.BlockSpec((ETYPE, HID), lambda i: (0, 0)),
            pl.BlockSpec((1, HID), lambda i: (0, 0)),
            pl.BlockSpec((1, 1), lambda i: (0, 0)),
        ],
        out_specs=[
            pl.BlockSpec((_BE, HID), lambda i: (i, 0)),
            pl.BlockSpec((1, 1, _BE), lambda i: (i, 0, 0)),
            pl.BlockSpec((8, 128), lambda i: (0, 0)),
        ],
        out_shape=[
            jax.ShapeDtypeStruct((E, HID), jnp.float32),
            jax.ShapeDtypeStruct((gbl, 1, _BE), jnp.float32),
            jax.ShapeDtypeStruct((8, 128), jnp.float32),
        ],
        scratch_shapes=[pltpu.SMEM((1, 1), jnp.float32)],
    )(mpre, et3, qb, wtil, c)


def _combine(parts, sp3, w2, b2, wa):
    return pl.pallas_call(
        _combine_body,
        grid=(N // _NBA,),
        in_specs=[
            pl.BlockSpec((NC, _NBA, HID), lambda i: (0, i, 0)),
            pl.BlockSpec((1, NC, _NBA), lambda i: (i, 0, 0)),
            pl.BlockSpec((HID, HID), lambda i: (0, 0)),
            pl.BlockSpec((1, HID), lambda i: (0, 0)),
            pl.BlockSpec((HID, HID), lambda i: (0, 0)),
        ],
        out_specs=pl.BlockSpec((_NBA, HID), lambda i: (i, 0)),
        out_shape=jax.ShapeDtypeStruct((N, HID), jnp.float32),
    )(parts, sp3, w2, b2, wa)


def _final(parts, sp3, w2, b2, fcw, fcb):
    return pl.pallas_call(
        _final_body,
        grid=(N // _NBA,),
        in_specs=[
            pl.BlockSpec((NC, _NBA, HID), lambda i: (0, i, 0)),
            pl.BlockSpec((1, NC, _NBA), lambda i: (i, 0, 0)),
            pl.BlockSpec((HID, HID), lambda i: (0, 0)),
            pl.BlockSpec((1, HID), lambda i: (0, 0)),
            pl.BlockSpec((HID, 16), lambda i: (0, 0)),
            pl.BlockSpec((1, 16), lambda i: (0, 0)),
        ],
        out_specs=pl.BlockSpec((1, 16), lambda i: (0, 0)),
        out_shape=jax.ShapeDtypeStruct((1, 16), jnp.float32),
        scratch_shapes=[pltpu.VMEM((1, HID), jnp.float32)],
    )(parts, sp3, w2, b2, fcw, fcb)


def kernel(x, node_type, edge_index, edge_type, landmark_mask, nt_table,
           in_proj_W, in_proj_b, gate_W1, gate_b1, gate_W2, gate_b2,
           c1_edge_emb, c1_msg_W1, c1_msg_b1, c1_msg_W2, c1_msg_b2,
           c1_att_W, c1_att_b, c2_edge_emb, c2_msg_W1, c2_msg_b1,
           c2_msg_W2, c2_msg_b2, c2_att_W, c2_att_b, fc_W, fc_b):
    f32 = jnp.float32
    node_type = node_type.astype(jnp.int32)
    edge_type = edge_type.astype(jnp.int32)
    src = edge_index[0].astype(jnp.int32)
    dst = edge_index[1].astype(jnp.int32)

    wx = in_proj_W[:HID]
    tn = jnp.dot(nt_table, in_proj_W[HID:], preferred_element_type=f32)
    b_in = in_proj_b[None, :]
    ts = jnp.dot(nt_table, gate_W1[0:8], preferred_element_type=f32)
    td = jnp.dot(nt_table, gate_W1[8:16], preferred_element_type=f32)
    us = gate_W1[16][None, :]
    ud = gate_W1[17][None, :]
    gb1 = gate_b1[None, :]
    w2t = gate_W2.T
    gb2 = gate_b2.reshape(1, 1)

    nt3 = node_type.reshape(N // _NBA, 1, _NBA)
    et3 = edge_type.reshape(E // _BE, 1, _BE)

    (_sc_gate_gather, _sc_row_gather, _sc_denom_cnt, _sc_denom_nocnt,
     _sc_coeff, _sc_aggregate) = _sc_kernels()

    nts, ntd, lms, lmd = _sc_gate_gather(node_type, landmark_mask, src, dst)
    gbl = E // _BE
    g = _gate(nts.reshape(gbl, 1, _BE), ntd.reshape(gbl, 1, _BE),
              lms.reshape(gbl, 1, _BE), lmd.reshape(gbl, 1, _BE),
              ts, td, us, ud, gb1, w2t, gb2).reshape(E)

    wa1 = c1_msg_W1[:HID]
    p1 = _prelude(x, nt3, wx, tn, b_in, wa1)

    def conv(p, edge_emb, msg_W1, msg_b1, msg_W2, msg_b2, att_W, att_b,
             do_cnt, cparts_prev):
        qb = jnp.dot(edge_emb, msg_W1[HID:], preferred_element_type=f32) \
            + msg_b1[None, :]
        wtil = jnp.dot(msg_W2, att_W, preferred_element_type=f32).T
        cconst = (jnp.dot(msg_b2[None, :], att_W, preferred_element_type=f32)
                  + att_b).reshape(1, 1)
        mpre = _sc_row_gather(p, src)
        r, a2, gmax8 = _edge_mlp(mpre, et3, qb, wtil, cconst)
        gmax16 = gmax8.reshape(-1)[:16]
        if do_cnt:
            eg, dparts, cparts = _sc_denom_cnt(a2.reshape(E), g, gmax16, dst)
        else:
            eg, dparts, _ = _sc_denom_nocnt(a2.reshape(E), g, gmax16, dst)
            cparts = cparts_prev
        cf = _sc_coeff(eg, dst, dparts, cparts)
        agg, sparts = _sc_aggregate(r, cf, dst)
        sp3 = sparts[:, :N].reshape(NC, N // _NBA, _NBA).transpose(1, 0, 2)
        return agg, sp3, cparts

    agg1, sp31, cparts = conv(p1, c1_edge_emb, c1_msg_W1, c1_msg_b1,
                              c1_msg_W2, c1_msg_b2, c1_att_W, c1_att_b,
                              True, None)
    p2 = _combine(agg1, sp31, c1_msg_W2, c1_msg_b2[None, :], c2_msg_W1[:HID])
    agg2, sp32, _ = conv(p2, c2_edge_emb, c2_msg_W1, c2_msg_b1, c2_msg_W2,
                         c2_msg_b2, c2_att_W, c2_att_b, False, cparts)
    return _final(agg2, sp32, c2_msg_W2, c2_msg_b2[None, :], fc_W,
                  fc_b.reshape(1, 16))
